# Initial kernel scaffold; baseline (speedup 1.0000x reference)
import jax
import jax.numpy as jnp
from jax.experimental import pallas as pl


def kernel(x_seq, edge_index, W1, b1, Wih, Whh, bih, bhh, W2, b2):
    raise NotImplementedError("write your pallas kernel here")


# speedup vs baseline: 72.8292x; 72.8292x over previous
import functools

import jax
import jax.numpy as jnp
from jax import lax
from jax.experimental import pallas as pl
from jax.experimental.pallas import tpu as pltpu
from jax.experimental.pallas import tpu_sc as plsc

F32 = jnp.float32

NC = 2
NS = 16
NW = NC * NS
CH = 125
RS = 80
BLK = 1024


def _mesh():
    return plsc.VectorSubcoreMesh(
        core_axis_name="c", subcore_axis_name="s", num_cores=NC, num_subcores=NS
    )


def _make_deg_kernel(NP, n_rows):
    rows_per_worker = n_rows // NW
    n_stages = rows_per_worker // RS
    rpt = NP // NS

    @functools.partial(
        pl.kernel,
        out_type=jax.ShapeDtypeStruct((NC * NP,), F32),
        mesh=_mesh(),
        scratch_types=[
            pltpu.VMEM((RS, CH), jnp.int32),
            pltpu.VMEM((128,), F32),
            pltpu.VMEM_SHARED((NP,), F32),
        ],
    )
    def deg_kernel(dst2d, zeros1, out, idx_v, ones_v, deg_sh):
        cid = lax.axis_index("c")
        sid = lax.axis_index("s")
        wid = cid * NS + sid
        base_row = wid * rows_per_worker
        r0 = sid * rpt
        pltpu.sync_copy(zeros1.at[pl.ds(r0, rpt)], deg_sh.at[pl.ds(r0, rpt)])
        for i in range(128 // 16):
            ones_v[pl.ds(i * 16, 16)] = jnp.ones((16,), F32)
        plsc.subcore_barrier()

        def stage(st, carry):
            pltpu.sync_copy(dst2d.at[pl.ds(base_row + st * RS, RS)], idx_v)

            def inner(j, c2):
                pltpu.sync_copy(ones_v.at[pl.ds(0, CH)],
                                deg_sh.at[idx_v.at[j]], add=True)
                return c2

            return lax.fori_loop(0, RS, inner, carry)

        lax.fori_loop(0, n_stages, stage, 0)
        plsc.subcore_barrier()
        pltpu.sync_copy(deg_sh.at[pl.ds(r0, rpt)],
                        out.at[pl.ds(cid * NP + r0, rpt)])

    return deg_kernel


def _make_time_agg_kernel(NP, T, n_rows):
    rows_per_worker = n_rows // NW
    n_stages = rows_per_worker // RS
    rpt = NP // NS

    @functools.partial(
        pl.kernel,
        out_type=jax.ShapeDtypeStruct((NC * T * NP,), F32),
        mesh=_mesh(),
        scratch_types=[
            pltpu.VMEM((RS, CH), jnp.int32),
            pltpu.VMEM((RS, CH), jnp.int32),
            pltpu.VMEM((T, CH), F32),
            [pltpu.VMEM_SHARED((NP,), F32) for _ in range(T)],
            pltpu.SemaphoreType.DMA,
        ],
    )
    def time_agg(*refs):
        xts = refs[0:T]
        src2d, dst2d, zeros1, out = refs[T:T + 4]
        src_v, dst_v, vals, accs, sem = refs[T + 4:]
        cid = lax.axis_index("c")
        sid = lax.axis_index("s")
        wid = cid * NS + sid
        base_row = wid * rows_per_worker
        r0 = sid * rpt
        for t in range(T):
            pltpu.sync_copy(zeros1.at[pl.ds(r0, rpt)],
                            accs[t].at[pl.ds(r0, rpt)])
        plsc.subcore_barrier()

        def stage(st, carry):
            row0 = base_row + st * RS
            pltpu.sync_copy(src2d.at[pl.ds(row0, RS)], src_v)
            pltpu.sync_copy(dst2d.at[pl.ds(row0, RS)], dst_v)

            def inner(j, c2):
                descs = [
                    pltpu.async_copy(xts[t].at[src_v.at[j]], vals.at[t], sem)
                    for t in range(T)
                ]
                for d in descs:
                    d.wait()
                for t in range(T):
                    pltpu.sync_copy(vals.at[t], accs[t].at[dst_v.at[j]],
                                    add=True)
                return c2

            return lax.fori_loop(0, RS, inner, carry)

        lax.fori_loop(0, n_stages, stage, 0)
        plsc.subcore_barrier()
        for t in range(T):
            pltpu.sync_copy(accs[t].at[pl.ds(r0, rpt)],
                            out.at[pl.ds((cid * T + t) * NP + r0, rpt)])

    return time_agg


def _make_scalar_agg_kernel(NP, n_rows):
    rows_per_worker = n_rows // NW
    n_stages = rows_per_worker // RS
    rpt = NP // NS

    @functools.partial(
        pl.kernel,
        out_type=jax.ShapeDtypeStruct((NC * NP,), F32),
        mesh=_mesh(),
        scratch_types=[
            pltpu.VMEM((RS, CH), jnp.int32),
            pltpu.VMEM((RS, CH), jnp.int32),
            pltpu.VMEM((CH,), F32),
            pltpu.VMEM_SHARED((NP,), F32),
            pltpu.SemaphoreType.DMA,
        ],
    )
    def scal_agg(ys, src2d, dst2d, zeros1, out,
                 src_v, dst_v, vals_v, b_sh, sem):
        cid = lax.axis_index("c")
        sid = lax.axis_index("s")
        wid = cid * NS + sid
        base_row = wid * rows_per_worker
        r0 = sid * rpt
        pltpu.sync_copy(zeros1.at[pl.ds(r0, rpt)], b_sh.at[pl.ds(r0, rpt)])
        plsc.subcore_barrier()

        def stage(st, carry):
            row0 = base_row + st * RS
            pltpu.sync_copy(src2d.at[pl.ds(row0, RS)], src_v)
            pltpu.sync_copy(dst2d.at[pl.ds(row0, RS)], dst_v)

            def inner(j, c2):
                pltpu.async_copy(ys.at[src_v.at[j]], vals_v, sem).wait()
                pltpu.sync_copy(vals_v, b_sh.at[dst_v.at[j]], add=True)
                return c2

            return lax.fori_loop(0, RS, inner, carry)

        lax.fori_loop(0, n_stages, stage, 0)
        plsc.subcore_barrier()
        pltpu.sync_copy(b_sh.at[pl.ds(r0, rpt)],
                        out.at[pl.ds(cid * NP + r0, rpt)])

    return scal_agg


def _prep_body(d, xT, dinv_o, xs_o):
    deg = d[0:1, :] + d[1:2, :] + 1.0
    dinv = lax.rsqrt(deg)
    dinv_o[...] = dinv
    xs_o[...] = dinv * xT[...]


def _make_gru_body(T, H):
    def _gru_body(aT, xsT, dinv, w1c, b1c, wih, whh, bic, bhc, w2r, ys_o):
        dv = dinv[...]
        s = dv * (aT[0:T, :] + aT[T:2 * T, :] + xsT[...])
        h = jnp.zeros((H, s.shape[1]), F32)
        for t in range(T):
            g = jax.nn.relu(w1c[...] * s[t:t + 1, :] + b1c[...])
            ui = jnp.dot(wih[...], g, preferred_element_type=F32) + bic[...]
            uh = jnp.dot(whh[...], h, preferred_element_type=F32) + bhc[...]
            r = jax.nn.sigmoid(ui[0:H] + uh[0:H])
            z = jax.nn.sigmoid(ui[H:2 * H] + uh[H:2 * H])
            nt = jnp.tanh(ui[2 * H:3 * H] + r * uh[2 * H:3 * H])
            h = (1.0 - z) * nt + z * h
        y = jnp.dot(w2r[...], h, preferred_element_type=F32)
        ys_o[...] = dv * y
    return _gru_body


def _out_body(b, ysc, dinv, b2s, o):
    o[...] = dinv[...] * (b[0:1, :] + b[1:2, :] + ysc[...]) + b2s[...]


def _row_spec(r):
    return pl.BlockSpec((r, BLK), lambda i: (0, i))


def _full_spec(shape):
    return pl.BlockSpec(shape, lambda i: tuple(0 for _ in shape))


def kernel(x_seq, edge_index, W1, b1, Wih, Whh, bih, bhh, W2, b2):
    N, T = x_seq.shape
    E = edge_index.shape[1]
    H = Whh.shape[1]
    NP = ((N + 2047) // 2048) * 2048
    nblocks = NP // BLK
    n_rows = E // CH

    xT = jnp.pad(x_seq.T, ((0, 0), (0, NP - N)))
    src2d = edge_index[0].reshape(n_rows, CH)
    dst2d = edge_index[1].reshape(n_rows, CH)
    zeros1 = jnp.zeros((NP,), F32)
    w1c = W1.reshape(H, 1)
    b1c = b1.reshape(H, 1)
    bic = bih.reshape(3 * H, 1)
    bhc = bhh.reshape(3 * H, 1)
    w2r = W2.reshape(1, H)
    b2s = b2.reshape(1, 1)

    deg2 = _make_deg_kernel(NP, n_rows)(dst2d, zeros1).reshape(NC, NP)

    dinv, xsT = pl.pallas_call(
        _prep_body,
        grid=(nblocks,),
        in_specs=[_row_spec(NC), _row_spec(T)],
        out_specs=[_row_spec(1), _row_spec(T)],
        out_shape=[jax.ShapeDtypeStruct((1, NP), F32),
                   jax.ShapeDtypeStruct((T, NP), F32)],
    )(deg2, xT)

    xts = [xsT[t] for t in range(T)]
    aT = _make_time_agg_kernel(NP, T, n_rows)(
        *xts, src2d, dst2d, zeros1).reshape(NC * T, NP)

    ysc = pl.pallas_call(
        _make_gru_body(T, H),
        grid=(nblocks,),
        in_specs=[_row_spec(NC * T), _row_spec(T), _row_spec(1),
                  _full_spec((H, 1)), _full_spec((H, 1)),
                  _full_spec((3 * H, H)), _full_spec((3 * H, H)),
                  _full_spec((3 * H, 1)), _full_spec((3 * H, 1)),
                  _full_spec((1, H))],
        out_specs=_row_spec(1),
        out_shape=jax.ShapeDtypeStruct((1, NP), F32),
    )(aT, xsT, dinv, w1c, b1c, Wih, Whh, bic, bhc, w2r)

    b2p = _make_scalar_agg_kernel(NP, n_rows)(
        ysc.reshape(NP), src2d, dst2d, zeros1).reshape(NC, NP)

    o = pl.pallas_call(
        _out_body,
        grid=(nblocks,),
        in_specs=[_row_spec(NC), _row_spec(1), _row_spec(1),
                  _full_spec((1, 1))],
        out_specs=_row_spec(1),
        out_shape=jax.ShapeDtypeStruct((1, NP), F32),
    )(b2p, ysc, dinv, b2s)

    return o[:, :N]


# speedup vs baseline: 94.4543x; 1.2969x over previous
import functools

import jax
import jax.numpy as jnp
from jax import lax
from jax.experimental import pallas as pl
from jax.experimental.pallas import tpu as pltpu
from jax.experimental.pallas import tpu_sc as plsc

F32 = jnp.float32

NC = 2
NS = 16
NW = NC * NS
CH = 125
RS = 80
BLK = 1024


def _mesh():
    return plsc.VectorSubcoreMesh(
        core_axis_name="c", subcore_axis_name="s", num_cores=NC, num_subcores=NS
    )


def _make_deg_kernel(NP, n_rows):
    rows_per_worker = n_rows // NW
    n_stages = rows_per_worker // RS
    rpt = NP // NS

    @functools.partial(
        pl.kernel,
        out_type=jax.ShapeDtypeStruct((NC * NP,), F32),
        mesh=_mesh(),
        scratch_types=[
            pltpu.VMEM((RS, CH), jnp.int32),
            pltpu.VMEM((128,), F32),
            pltpu.VMEM_SHARED((NP,), F32),
        ],
    )
    def deg_kernel(dst2d, zeros1, out, idx_v, ones_v, deg_sh):
        cid = lax.axis_index("c")
        sid = lax.axis_index("s")
        wid = cid * NS + sid
        base_row = wid * rows_per_worker
        r0 = sid * rpt
        pltpu.sync_copy(zeros1.at[pl.ds(r0, rpt)], deg_sh.at[pl.ds(r0, rpt)])
        for i in range(128 // 16):
            ones_v[pl.ds(i * 16, 16)] = jnp.ones((16,), F32)
        plsc.subcore_barrier()

        def stage(st, carry):
            pltpu.sync_copy(dst2d.at[pl.ds(base_row + st * RS, RS)], idx_v)

            def inner(j, c2):
                pltpu.sync_copy(ones_v.at[pl.ds(0, CH)],
                                deg_sh.at[idx_v.at[j]], add=True)
                return c2

            return lax.fori_loop(0, RS, inner, carry)

        lax.fori_loop(0, n_stages, stage, 0)
        plsc.subcore_barrier()
        pltpu.sync_copy(deg_sh.at[pl.ds(r0, rpt)],
                        out.at[pl.ds(cid * NP + r0, rpt)])

    return deg_kernel


def _make_time_agg_kernel(NP, T, n_rows):
    rows_per_worker = n_rows // NW
    n_stages = rows_per_worker // RS
    rpt = NP // NS

    @functools.partial(
        pl.kernel,
        out_type=jax.ShapeDtypeStruct((NC * T * NP,), F32),
        mesh=_mesh(),
        scratch_types=[
            pltpu.VMEM((RS, CH), jnp.int32),
            pltpu.VMEM((RS, CH), jnp.int32),
            pltpu.VMEM((T, CH), F32),
            pltpu.VMEM((T * CH,), F32),
            [pltpu.VMEM_SHARED((NP,), F32) for _ in range(T)],
            pltpu.SemaphoreType.DMA,
            pltpu.SemaphoreType.DMA,
        ],
    )
    def time_agg(*refs):
        xts = refs[0:T]
        src2d, dst2d, zeros1, out = refs[T:T + 4]
        src_v, dst_v, vals, drain_v, accs, gsem, ssem = refs[T + 4:]
        cid = lax.axis_index("c")
        sid = lax.axis_index("s")
        wid = cid * NS + sid
        base_row = wid * rows_per_worker
        r0 = sid * rpt
        for t in range(T):
            pltpu.sync_copy(zeros1.at[pl.ds(r0, rpt)],
                            accs[t].at[pl.ds(r0, rpt)])
        plsc.subcore_barrier()

        def _drain_scatters():
            pltpu.make_async_copy(zeros1.at[pl.ds(0, T * CH)], drain_v,
                                  ssem).wait()

        def stage(st, carry):
            row0 = base_row + st * RS
            pltpu.sync_copy(src2d.at[pl.ds(row0, RS)], src_v)
            pltpu.sync_copy(dst2d.at[pl.ds(row0, RS)], dst_v)

            def inner(j, c2):
                descs = [
                    pltpu.async_copy(xts[t].at[src_v.at[j]], vals.at[t], gsem)
                    for t in range(T)
                ]

                @pl.when(j > 0)
                def _():
                    _drain_scatters()

                for d in descs:
                    d.wait()
                for t in range(T):
                    pltpu.async_copy(vals.at[t], accs[t].at[dst_v.at[j]],
                                    ssem, add=True)
                return c2

            r = lax.fori_loop(0, RS, inner, carry)
            _drain_scatters()
            return r

        lax.fori_loop(0, n_stages, stage, 0)
        plsc.subcore_barrier()
        for t in range(T):
            pltpu.sync_copy(accs[t].at[pl.ds(r0, rpt)],
                            out.at[pl.ds((cid * T + t) * NP + r0, rpt)])

    return time_agg


def _make_scalar_agg_kernel(NP, n_rows):
    rows_per_worker = n_rows // NW
    n_stages = rows_per_worker // RS
    rpt = NP // NS

    @functools.partial(
        pl.kernel,
        out_type=jax.ShapeDtypeStruct((NC * NP,), F32),
        mesh=_mesh(),
        scratch_types=[
            pltpu.VMEM((RS, CH), jnp.int32),
            pltpu.VMEM((RS, CH), jnp.int32),
            pltpu.VMEM((CH,), F32),
            pltpu.VMEM((CH,), F32),
            pltpu.VMEM_SHARED((NP,), F32),
            pltpu.SemaphoreType.DMA,
            pltpu.SemaphoreType.DMA,
        ],
    )
    def scal_agg(ys, src2d, dst2d, zeros1, out,
                 src_v, dst_v, vals_v, drain_v, b_sh, gsem, ssem):
        cid = lax.axis_index("c")
        sid = lax.axis_index("s")
        wid = cid * NS + sid
        base_row = wid * rows_per_worker
        r0 = sid * rpt
        pltpu.sync_copy(zeros1.at[pl.ds(r0, rpt)], b_sh.at[pl.ds(r0, rpt)])
        plsc.subcore_barrier()

        def _drain_scatter():
            pltpu.make_async_copy(zeros1.at[pl.ds(0, CH)], drain_v,
                                  ssem).wait()

        def stage(st, carry):
            row0 = base_row + st * RS
            pltpu.sync_copy(src2d.at[pl.ds(row0, RS)], src_v)
            pltpu.sync_copy(dst2d.at[pl.ds(row0, RS)], dst_v)

            def inner(j, c2):
                g = pltpu.async_copy(ys.at[src_v.at[j]], vals_v, gsem)

                @pl.when(j > 0)
                def _():
                    _drain_scatter()

                g.wait()
                pltpu.async_copy(vals_v, b_sh.at[dst_v.at[j]], ssem, add=True)
                return c2

            r = lax.fori_loop(0, RS, inner, carry)
            _drain_scatter()
            return r

        lax.fori_loop(0, n_stages, stage, 0)
        plsc.subcore_barrier()
        pltpu.sync_copy(b_sh.at[pl.ds(r0, rpt)],
                        out.at[pl.ds(cid * NP + r0, rpt)])

    return scal_agg


def _prep_body(d, xT, dinv_o, xs_o):
    deg = d[0:1, :] + d[1:2, :] + 1.0
    dinv = lax.rsqrt(deg)
    dinv_o[...] = dinv
    xs_o[...] = dinv * xT[...]


def _make_gru_body(T, H):
    def _gru_body(aT, xsT, dinv, w1c, b1c, wih, whh, bic, bhc, w2r, ys_o):
        dv = dinv[...]
        s = dv * (aT[0:T, :] + aT[T:2 * T, :] + xsT[...])
        h = jnp.zeros((H, s.shape[1]), F32)
        for t in range(T):
            g = jax.nn.relu(w1c[...] * s[t:t + 1, :] + b1c[...])
            ui = jnp.dot(wih[...], g, preferred_element_type=F32) + bic[...]
            uh = jnp.dot(whh[...], h, preferred_element_type=F32) + bhc[...]
            r = jax.nn.sigmoid(ui[0:H] + uh[0:H])
            z = jax.nn.sigmoid(ui[H:2 * H] + uh[H:2 * H])
            nt = jnp.tanh(ui[2 * H:3 * H] + r * uh[2 * H:3 * H])
            h = (1.0 - z) * nt + z * h
        y = jnp.dot(w2r[...], h, preferred_element_type=F32)
        ys_o[...] = dv * y
    return _gru_body


def _out_body(b, ysc, dinv, b2s, o):
    o[...] = dinv[...] * (b[0:1, :] + b[1:2, :] + ysc[...]) + b2s[...]


def _row_spec(r):
    return pl.BlockSpec((r, BLK), lambda i: (0, i))


def _full_spec(shape):
    return pl.BlockSpec(shape, lambda i: tuple(0 for _ in shape))


def kernel(x_seq, edge_index, W1, b1, Wih, Whh, bih, bhh, W2, b2):
    N, T = x_seq.shape
    E = edge_index.shape[1]
    H = Whh.shape[1]
    NP = ((N + 2047) // 2048) * 2048
    nblocks = NP // BLK
    n_rows = E // CH

    xT = jnp.pad(x_seq.T, ((0, 0), (0, NP - N)))
    src2d = edge_index[0].reshape(n_rows, CH)
    dst2d = edge_index[1].reshape(n_rows, CH)
    zeros1 = jnp.zeros((NP,), F32)
    w1c = W1.reshape(H, 1)
    b1c = b1.reshape(H, 1)
    bic = bih.reshape(3 * H, 1)
    bhc = bhh.reshape(3 * H, 1)
    w2r = W2.reshape(1, H)
    b2s = b2.reshape(1, 1)

    deg2 = _make_deg_kernel(NP, n_rows)(dst2d, zeros1).reshape(NC, NP)

    dinv, xsT = pl.pallas_call(
        _prep_body,
        grid=(nblocks,),
        in_specs=[_row_spec(NC), _row_spec(T)],
        out_specs=[_row_spec(1), _row_spec(T)],
        out_shape=[jax.ShapeDtypeStruct((1, NP), F32),
                   jax.ShapeDtypeStruct((T, NP), F32)],
    )(deg2, xT)

    xts = [xsT[t] for t in range(T)]
    aT = _make_time_agg_kernel(NP, T, n_rows)(
        *xts, src2d, dst2d, zeros1).reshape(NC * T, NP)

    ysc = pl.pallas_call(
        _make_gru_body(T, H),
        grid=(nblocks,),
        in_specs=[_row_spec(NC * T), _row_spec(T), _row_spec(1),
                  _full_spec((H, 1)), _full_spec((H, 1)),
                  _full_spec((3 * H, H)), _full_spec((3 * H, H)),
                  _full_spec((3 * H, 1)), _full_spec((3 * H, 1)),
                  _full_spec((1, H))],
        out_specs=_row_spec(1),
        out_shape=jax.ShapeDtypeStruct((1, NP), F32),
    )(aT, xsT, dinv, w1c, b1c, Wih, Whh, bic, bhc, w2r)

    b2p = _make_scalar_agg_kernel(NP, n_rows)(
        ysc.reshape(NP), src2d, dst2d, zeros1).reshape(NC, NP)

    o = pl.pallas_call(
        _out_body,
        grid=(nblocks,),
        in_specs=[_row_spec(NC), _row_spec(1), _row_spec(1),
                  _full_spec((1, 1))],
        out_specs=_row_spec(1),
        out_shape=jax.ShapeDtypeStruct((1, NP), F32),
    )(b2p, ysc, dinv, b2s)

    return o[:, :N]


# speedup vs baseline: 107.1352x; 1.1343x over previous
import functools

import jax
import jax.numpy as jnp
from jax import lax
from jax.experimental import pallas as pl
from jax.experimental.pallas import tpu as pltpu
from jax.experimental.pallas import tpu_sc as plsc

F32 = jnp.float32

NC = 2
NS = 16
NW = NC * NS
CH = 125
RS = 80
BLK = 1024


def _mesh():
    return plsc.VectorSubcoreMesh(
        core_axis_name="c", subcore_axis_name="s", num_cores=NC, num_subcores=NS
    )


def _make_deg_kernel(NP, n_rows):
    rows_per_worker = n_rows // NW
    n_stages = rows_per_worker // RS
    rpt = NP // NS

    @functools.partial(
        pl.kernel,
        out_type=jax.ShapeDtypeStruct((NC * NP,), F32),
        mesh=_mesh(),
        scratch_types=[
            pltpu.VMEM((RS, CH), jnp.int32),
            pltpu.VMEM((128,), F32),
            pltpu.VMEM_SHARED((NP,), F32),
        ],
    )
    def deg_kernel(dst2d, zeros1, out, idx_v, ones_v, deg_sh):
        cid = lax.axis_index("c")
        sid = lax.axis_index("s")
        wid = cid * NS + sid
        base_row = wid * rows_per_worker
        r0 = sid * rpt
        pltpu.sync_copy(zeros1.at[pl.ds(r0, rpt)], deg_sh.at[pl.ds(r0, rpt)])
        for i in range(128 // 16):
            ones_v[pl.ds(i * 16, 16)] = jnp.ones((16,), F32)
        plsc.subcore_barrier()

        def stage(st, carry):
            pltpu.sync_copy(dst2d.at[pl.ds(base_row + st * RS, RS)], idx_v)

            def inner(j, c2):
                pltpu.sync_copy(ones_v.at[pl.ds(0, CH)],
                                deg_sh.at[idx_v.at[j]], add=True)
                return c2

            return lax.fori_loop(0, RS, inner, carry)

        lax.fori_loop(0, n_stages, stage, 0)
        plsc.subcore_barrier()
        pltpu.sync_copy(deg_sh.at[pl.ds(r0, rpt)],
                        out.at[pl.ds(cid * NP + r0, rpt)])

    return deg_kernel


def _make_time_agg_kernel(NP, T, n_rows):
    rows_per_worker = n_rows // NW
    n_stages = rows_per_worker // RS
    rpt = NP // NS

    @functools.partial(
        pl.kernel,
        out_type=jax.ShapeDtypeStruct((NC * T * NP,), F32),
        mesh=_mesh(),
        scratch_types=[
            pltpu.VMEM((RS, CH), jnp.int32),
            pltpu.VMEM((RS, CH), jnp.int32),
            pltpu.VMEM((2, T, CH), F32),
            pltpu.VMEM((T * CH,), F32),
            [pltpu.VMEM_SHARED((NP,), F32) for _ in range(T)],
            pltpu.SemaphoreType.DMA,
            pltpu.SemaphoreType.DMA,
        ],
    )
    def time_agg(*refs):
        xts = refs[0:T]
        src2d, dst2d, zeros1, out = refs[T:T + 4]
        src_v, dst_v, vals, drain_v, accs, gsem, ssem = refs[T + 4:]
        cid = lax.axis_index("c")
        sid = lax.axis_index("s")
        wid = cid * NS + sid
        base_row = wid * rows_per_worker
        r0 = sid * rpt
        for t in range(T):
            pltpu.sync_copy(zeros1.at[pl.ds(r0, rpt)],
                            accs[t].at[pl.ds(r0, rpt)])
        plsc.subcore_barrier()

        row_bytes = 4 * CH

        def _drain(sem, nbytes):
            pltpu.make_async_copy(zeros1.at[pl.ds(0, nbytes // 4)],
                                  drain_v.at[pl.ds(0, nbytes // 4)],
                                  sem).wait()

        def stage(st, carry):
            row0 = base_row + st * RS
            pltpu.sync_copy(src2d.at[pl.ds(row0, RS)], src_v)
            pltpu.sync_copy(dst2d.at[pl.ds(row0, RS)], dst_v)

            for t in range(T):
                pltpu.async_copy(xts[t].at[src_v.at[0]], vals.at[0, t], gsem)

            def inner(j, c2):
                p = lax.rem(j, 2)

                @pl.when(j < RS - 1)
                def _():
                    for t in range(T):
                        pltpu.async_copy(xts[t].at[src_v.at[j + 1]],
                                         vals.at[1 - p, t], gsem)

                @pl.when(j > 0)
                def _():
                    _drain(ssem, T * row_bytes)

                for t in range(T):
                    _drain(gsem, row_bytes)
                    pltpu.async_copy(vals.at[p, t], accs[t].at[dst_v.at[j]],
                                     ssem, add=True)
                return c2

            r = lax.fori_loop(0, RS, inner, carry)
            _drain(ssem, T * row_bytes)
            return r

        lax.fori_loop(0, n_stages, stage, 0)
        plsc.subcore_barrier()
        for t in range(T):
            pltpu.sync_copy(accs[t].at[pl.ds(r0, rpt)],
                            out.at[pl.ds((cid * T + t) * NP + r0, rpt)])

    return time_agg


def _make_scalar_agg_kernel(NP, n_rows):
    rows_per_worker = n_rows // NW
    n_stages = rows_per_worker // RS
    rpt = NP // NS

    @functools.partial(
        pl.kernel,
        out_type=jax.ShapeDtypeStruct((NC * NP,), F32),
        mesh=_mesh(),
        scratch_types=[
            pltpu.VMEM((RS, CH), jnp.int32),
            pltpu.VMEM((RS, CH), jnp.int32),
            pltpu.VMEM((2, CH), F32),
            pltpu.VMEM((CH,), F32),
            pltpu.VMEM_SHARED((NP,), F32),
            pltpu.SemaphoreType.DMA,
            pltpu.SemaphoreType.DMA,
        ],
    )
    def scal_agg(ys, src2d, dst2d, zeros1, out,
                 src_v, dst_v, vals, drain_v, b_sh, gsem, ssem):
        cid = lax.axis_index("c")
        sid = lax.axis_index("s")
        wid = cid * NS + sid
        base_row = wid * rows_per_worker
        r0 = sid * rpt
        pltpu.sync_copy(zeros1.at[pl.ds(r0, rpt)], b_sh.at[pl.ds(r0, rpt)])
        plsc.subcore_barrier()

        row_bytes = 4 * CH

        def _drain(sem, nbytes):
            pltpu.make_async_copy(zeros1.at[pl.ds(0, nbytes // 4)],
                                  drain_v.at[pl.ds(0, nbytes // 4)],
                                  sem).wait()

        def stage(st, carry):
            row0 = base_row + st * RS
            pltpu.sync_copy(src2d.at[pl.ds(row0, RS)], src_v)
            pltpu.sync_copy(dst2d.at[pl.ds(row0, RS)], dst_v)
            pltpu.async_copy(ys.at[src_v.at[0]], vals.at[0], gsem)

            def inner(j, c2):
                p = lax.rem(j, 2)

                @pl.when(j < RS - 1)
                def _():
                    pltpu.async_copy(ys.at[src_v.at[j + 1]],
                                     vals.at[1 - p], gsem)

                @pl.when(j > 0)
                def _():
                    _drain(ssem, row_bytes)

                _drain(gsem, row_bytes)
                pltpu.async_copy(vals.at[p], b_sh.at[dst_v.at[j]],
                                 ssem, add=True)
                return c2

            r = lax.fori_loop(0, RS, inner, carry)
            _drain(ssem, row_bytes)
            return r

        lax.fori_loop(0, n_stages, stage, 0)
        plsc.subcore_barrier()
        pltpu.sync_copy(b_sh.at[pl.ds(r0, rpt)],
                        out.at[pl.ds(cid * NP + r0, rpt)])

    return scal_agg


def _prep_body(d, xT, dinv_o, xs_o):
    deg = d[0:1, :] + d[1:2, :] + 1.0
    dinv = lax.rsqrt(deg)
    dinv_o[...] = dinv
    xs_o[...] = dinv * xT[...]


def _make_gru_body(T, H):
    def _gru_body(aT, xsT, dinv, w1c, b1c, wih, whh, bic, bhc, w2r, ys_o):
        dv = dinv[...]
        s = dv * (aT[0:T, :] + aT[T:2 * T, :] + xsT[...])
        h = jnp.zeros((H, s.shape[1]), F32)
        for t in range(T):
            g = jax.nn.relu(w1c[...] * s[t:t + 1, :] + b1c[...])
            ui = jnp.dot(wih[...], g, preferred_element_type=F32) + bic[...]
            uh = jnp.dot(whh[...], h, preferred_element_type=F32) + bhc[...]
            r = jax.nn.sigmoid(ui[0:H] + uh[0:H])
            z = jax.nn.sigmoid(ui[H:2 * H] + uh[H:2 * H])
            nt = jnp.tanh(ui[2 * H:3 * H] + r * uh[2 * H:3 * H])
            h = (1.0 - z) * nt + z * h
        y = jnp.dot(w2r[...], h, preferred_element_type=F32)
        ys_o[...] = dv * y
    return _gru_body


def _out_body(b, ysc, dinv, b2s, o):
    o[...] = dinv[...] * (b[0:1, :] + b[1:2, :] + ysc[...]) + b2s[...]


def _row_spec(r):
    return pl.BlockSpec((r, BLK), lambda i: (0, i))


def _full_spec(shape):
    return pl.BlockSpec(shape, lambda i: tuple(0 for _ in shape))


def kernel(x_seq, edge_index, W1, b1, Wih, Whh, bih, bhh, W2, b2):
    N, T = x_seq.shape
    E = edge_index.shape[1]
    H = Whh.shape[1]
    NP = ((N + 2047) // 2048) * 2048
    nblocks = NP // BLK
    n_rows = E // CH

    xT = jnp.pad(x_seq.T, ((0, 0), (0, NP - N)))
    src2d = edge_index[0].reshape(n_rows, CH)
    dst2d = edge_index[1].reshape(n_rows, CH)
    zeros1 = jnp.zeros((NP,), F32)
    w1c = W1.reshape(H, 1)
    b1c = b1.reshape(H, 1)
    bic = bih.reshape(3 * H, 1)
    bhc = bhh.reshape(3 * H, 1)
    w2r = W2.reshape(1, H)
    b2s = b2.reshape(1, 1)

    deg2 = _make_deg_kernel(NP, n_rows)(dst2d, zeros1).reshape(NC, NP)

    dinv, xsT = pl.pallas_call(
        _prep_body,
        grid=(nblocks,),
        in_specs=[_row_spec(NC), _row_spec(T)],
        out_specs=[_row_spec(1), _row_spec(T)],
        out_shape=[jax.ShapeDtypeStruct((1, NP), F32),
                   jax.ShapeDtypeStruct((T, NP), F32)],
    )(deg2, xT)

    xts = [xsT[t] for t in range(T)]
    aT = _make_time_agg_kernel(NP, T, n_rows)(
        *xts, src2d, dst2d, zeros1).reshape(NC * T, NP)

    ysc = pl.pallas_call(
        _make_gru_body(T, H),
        grid=(nblocks,),
        in_specs=[_row_spec(NC * T), _row_spec(T), _row_spec(1),
                  _full_spec((H, 1)), _full_spec((H, 1)),
                  _full_spec((3 * H, H)), _full_spec((3 * H, H)),
                  _full_spec((3 * H, 1)), _full_spec((3 * H, 1)),
                  _full_spec((1, H))],
        out_specs=_row_spec(1),
        out_shape=jax.ShapeDtypeStruct((1, NP), F32),
    )(aT, xsT, dinv, w1c, b1c, Wih, Whh, bic, bhc, w2r)

    b2p = _make_scalar_agg_kernel(NP, n_rows)(
        ysc.reshape(NP), src2d, dst2d, zeros1).reshape(NC, NP)

    o = pl.pallas_call(
        _out_body,
        grid=(nblocks,),
        in_specs=[_row_spec(NC), _row_spec(1), _row_spec(1),
                  _full_spec((1, 1))],
        out_specs=_row_spec(1),
        out_shape=jax.ShapeDtypeStruct((1, NP), F32),
    )(b2p, ysc, dinv, b2s)

    return o[:, :N]


# speedup vs baseline: 110.2741x; 1.0293x over previous
import functools

import jax
import jax.numpy as jnp
from jax import lax
from jax.experimental import pallas as pl
from jax.experimental.pallas import tpu as pltpu
from jax.experimental.pallas import tpu_sc as plsc

F32 = jnp.float32

NC = 2
NS = 16
NW = NC * NS
CH = 125
RS = 80
BLK = 1024


def _mesh():
    return plsc.VectorSubcoreMesh(
        core_axis_name="c", subcore_axis_name="s", num_cores=NC, num_subcores=NS
    )


def _make_deg_kernel(NP, n_rows):
    rows_per_worker = n_rows // NW
    n_stages = rows_per_worker // RS
    rpt = NP // NS

    @functools.partial(
        pl.kernel,
        out_type=jax.ShapeDtypeStruct((NC * NP,), F32),
        mesh=_mesh(),
        scratch_types=[
            pltpu.VMEM((RS, CH), jnp.int32),
            pltpu.VMEM((128,), F32),
            pltpu.VMEM_SHARED((NP,), F32),
        ],
    )
    def deg_kernel(dst2d, zeros1, out, idx_v, ones_v, deg_sh):
        cid = lax.axis_index("c")
        sid = lax.axis_index("s")
        wid = cid * NS + sid
        base_row = wid * rows_per_worker
        r0 = sid * rpt
        pltpu.sync_copy(zeros1.at[pl.ds(r0, rpt)], deg_sh.at[pl.ds(r0, rpt)])
        for i in range(128 // 16):
            ones_v[pl.ds(i * 16, 16)] = jnp.ones((16,), F32)
        plsc.subcore_barrier()

        def stage(st, carry):
            pltpu.sync_copy(dst2d.at[pl.ds(base_row + st * RS, RS)], idx_v)

            def inner(j, c2):
                pltpu.sync_copy(ones_v.at[pl.ds(0, CH)],
                                deg_sh.at[idx_v.at[j]], add=True)
                return c2

            return lax.fori_loop(0, RS, inner, carry)

        lax.fori_loop(0, n_stages, stage, 0)
        plsc.subcore_barrier()
        pltpu.sync_copy(deg_sh.at[pl.ds(r0, rpt)],
                        out.at[pl.ds(cid * NP + r0, rpt)])

    return deg_kernel


def _make_time_agg_kernel(NP, T, n_rows):
    rows_per_worker = n_rows // NW
    n_stages = rows_per_worker // RS
    rpt = NP // NS

    @functools.partial(
        pl.kernel,
        out_type=jax.ShapeDtypeStruct((NC * T * NP,), F32),
        mesh=_mesh(),
        scratch_types=[
            pltpu.VMEM((RS, CH), jnp.int32),
            pltpu.VMEM((RS, CH), jnp.int32),
            pltpu.VMEM((2, T, CH), F32),
            pltpu.VMEM((T * CH,), F32),
            [pltpu.VMEM_SHARED((NP,), F32) for _ in range(T)],
            pltpu.SemaphoreType.DMA,
            pltpu.SemaphoreType.DMA,
        ],
    )
    def time_agg(*refs):
        xts = refs[0:T]
        src2d, dst2d, zeros1, out = refs[T:T + 4]
        src_v, dst_v, vals, drain_v, accs, gsem, ssem = refs[T + 4:]
        cid = lax.axis_index("c")
        sid = lax.axis_index("s")
        wid = cid * NS + sid
        base_row = wid * rows_per_worker
        r0 = sid * rpt
        for t in range(T):
            pltpu.sync_copy(zeros1.at[pl.ds(r0, rpt)],
                            accs[t].at[pl.ds(r0, rpt)])
        plsc.subcore_barrier()

        row_bytes = 4 * CH

        def _drain(sem, nbytes):
            pltpu.make_async_copy(zeros1.at[pl.ds(0, nbytes // 4)],
                                  drain_v.at[pl.ds(0, nbytes // 4)],
                                  sem).wait()

        def stage(st, carry):
            row0 = base_row + st * RS
            pltpu.sync_copy(src2d.at[pl.ds(row0, RS)], src_v)
            pltpu.sync_copy(dst2d.at[pl.ds(row0, RS)], dst_v)

            for t in range(T):
                pltpu.async_copy(xts[t].at[src_v.at[0]], vals.at[0, t], gsem)

            def inner(j, c2):
                p = lax.rem(j, 2)

                @pl.when(j < RS - 1)
                def _():
                    for t in range(T):
                        pltpu.async_copy(xts[t].at[src_v.at[j + 1]],
                                         vals.at[1 - p, t], gsem)

                @pl.when(j > 0)
                def _():
                    _drain(ssem, T * row_bytes)

                _drain(gsem, T * row_bytes)
                for t in range(T):
                    pltpu.async_copy(vals.at[p, t], accs[t].at[dst_v.at[j]],
                                     ssem, add=True)
                return c2

            r = lax.fori_loop(0, RS, inner, carry)
            _drain(ssem, T * row_bytes)
            return r

        lax.fori_loop(0, n_stages, stage, 0)
        plsc.subcore_barrier()
        for t in range(T):
            pltpu.sync_copy(accs[t].at[pl.ds(r0, rpt)],
                            out.at[pl.ds((cid * T + t) * NP + r0, rpt)])

    return time_agg


def _make_scalar_agg_kernel(NP, n_rows):
    rows_per_worker = n_rows // NW
    n_stages = rows_per_worker // RS
    rpt = NP // NS

    @functools.partial(
        pl.kernel,
        out_type=jax.ShapeDtypeStruct((NC * NP,), F32),
        mesh=_mesh(),
        scratch_types=[
            pltpu.VMEM((RS, CH), jnp.int32),
            pltpu.VMEM((RS, CH), jnp.int32),
            pltpu.VMEM((2, CH), F32),
            pltpu.VMEM((CH,), F32),
            pltpu.VMEM_SHARED((NP,), F32),
            pltpu.SemaphoreType.DMA,
            pltpu.SemaphoreType.DMA,
        ],
    )
    def scal_agg(ys, src2d, dst2d, zeros1, out,
                 src_v, dst_v, vals, drain_v, b_sh, gsem, ssem):
        cid = lax.axis_index("c")
        sid = lax.axis_index("s")
        wid = cid * NS + sid
        base_row = wid * rows_per_worker
        r0 = sid * rpt
        pltpu.sync_copy(zeros1.at[pl.ds(r0, rpt)], b_sh.at[pl.ds(r0, rpt)])
        plsc.subcore_barrier()

        row_bytes = 4 * CH

        def _drain(sem, nbytes):
            pltpu.make_async_copy(zeros1.at[pl.ds(0, nbytes // 4)],
                                  drain_v.at[pl.ds(0, nbytes // 4)],
                                  sem).wait()

        def stage(st, carry):
            row0 = base_row + st * RS
            pltpu.sync_copy(src2d.at[pl.ds(row0, RS)], src_v)
            pltpu.sync_copy(dst2d.at[pl.ds(row0, RS)], dst_v)
            pltpu.async_copy(ys.at[src_v.at[0]], vals.at[0], gsem)

            def inner(j, c2):
                p = lax.rem(j, 2)

                @pl.when(j < RS - 1)
                def _():
                    pltpu.async_copy(ys.at[src_v.at[j + 1]],
                                     vals.at[1 - p], gsem)

                @pl.when(j > 0)
                def _():
                    _drain(ssem, row_bytes)

                _drain(gsem, row_bytes)
                pltpu.async_copy(vals.at[p], b_sh.at[dst_v.at[j]],
                                 ssem, add=True)
                return c2

            r = lax.fori_loop(0, RS, inner, carry)
            _drain(ssem, row_bytes)
            return r

        lax.fori_loop(0, n_stages, stage, 0)
        plsc.subcore_barrier()
        pltpu.sync_copy(b_sh.at[pl.ds(r0, rpt)],
                        out.at[pl.ds(cid * NP + r0, rpt)])

    return scal_agg


def _prep_body(d, xT, dinv_o, xs_o):
    deg = d[0:1, :] + d[1:2, :] + 1.0
    dinv = lax.rsqrt(deg)
    dinv_o[...] = dinv
    xs_o[...] = dinv * xT[...]


def _make_gru_body(T, H):
    def _gru_body(aT, xsT, dinv, w1c, b1c, wih, whh, bic, bhc, w2r, ys_o):
        dv = dinv[...]
        s = dv * (aT[0:T, :] + aT[T:2 * T, :] + xsT[...])
        h = jnp.zeros((H, s.shape[1]), F32)
        for t in range(T):
            g = jax.nn.relu(w1c[...] * s[t:t + 1, :] + b1c[...])
            ui = jnp.dot(wih[...], g, preferred_element_type=F32) + bic[...]
            uh = jnp.dot(whh[...], h, preferred_element_type=F32) + bhc[...]
            r = jax.nn.sigmoid(ui[0:H] + uh[0:H])
            z = jax.nn.sigmoid(ui[H:2 * H] + uh[H:2 * H])
            nt = jnp.tanh(ui[2 * H:3 * H] + r * uh[2 * H:3 * H])
            h = (1.0 - z) * nt + z * h
        y = jnp.dot(w2r[...], h, preferred_element_type=F32)
        ys_o[...] = dv * y
    return _gru_body


def _out_body(b, ysc, dinv, b2s, o):
    o[...] = dinv[...] * (b[0:1, :] + b[1:2, :] + ysc[...]) + b2s[...]


def _row_spec(r):
    return pl.BlockSpec((r, BLK), lambda i: (0, i))


def _full_spec(shape):
    return pl.BlockSpec(shape, lambda i: tuple(0 for _ in shape))


def kernel(x_seq, edge_index, W1, b1, Wih, Whh, bih, bhh, W2, b2):
    N, T = x_seq.shape
    E = edge_index.shape[1]
    H = Whh.shape[1]
    NP = ((N + 2047) // 2048) * 2048
    nblocks = NP // BLK
    n_rows = E // CH

    xT = jnp.pad(x_seq.T, ((0, 0), (0, NP - N)))
    src2d = edge_index[0].reshape(n_rows, CH)
    dst2d = edge_index[1].reshape(n_rows, CH)
    zeros1 = jnp.zeros((NP,), F32)
    w1c = W1.reshape(H, 1)
    b1c = b1.reshape(H, 1)
    bic = bih.reshape(3 * H, 1)
    bhc = bhh.reshape(3 * H, 1)
    w2r = W2.reshape(1, H)
    b2s = b2.reshape(1, 1)

    deg2 = _make_deg_kernel(NP, n_rows)(dst2d, zeros1).reshape(NC, NP)

    dinv, xsT = pl.pallas_call(
        _prep_body,
        grid=(nblocks,),
        in_specs=[_row_spec(NC), _row_spec(T)],
        out_specs=[_row_spec(1), _row_spec(T)],
        out_shape=[jax.ShapeDtypeStruct((1, NP), F32),
                   jax.ShapeDtypeStruct((T, NP), F32)],
    )(deg2, xT)

    xts = [xsT[t] for t in range(T)]
    aT = _make_time_agg_kernel(NP, T, n_rows)(
        *xts, src2d, dst2d, zeros1).reshape(NC * T, NP)

    ysc = pl.pallas_call(
        _make_gru_body(T, H),
        grid=(nblocks,),
        in_specs=[_row_spec(NC * T), _row_spec(T), _row_spec(1),
                  _full_spec((H, 1)), _full_spec((H, 1)),
                  _full_spec((3 * H, H)), _full_spec((3 * H, H)),
                  _full_spec((3 * H, 1)), _full_spec((3 * H, 1)),
                  _full_spec((1, H))],
        out_specs=_row_spec(1),
        out_shape=jax.ShapeDtypeStruct((1, NP), F32),
    )(aT, xsT, dinv, w1c, b1c, Wih, Whh, bic, bhc, w2r)

    b2p = _make_scalar_agg_kernel(NP, n_rows)(
        ysc.reshape(NP), src2d, dst2d, zeros1).reshape(NC, NP)

    o = pl.pallas_call(
        _out_body,
        grid=(nblocks,),
        in_specs=[_row_spec(NC), _row_spec(1), _row_spec(1),
                  _full_spec((1, 1))],
        out_specs=_row_spec(1),
        out_shape=jax.ShapeDtypeStruct((1, NP), F32),
    )(b2p, ysc, dinv, b2s)

    return o[:, :N]


# speedup vs baseline: 110.3866x; 1.0010x over previous
import functools

import jax
import jax.numpy as jnp
from jax import lax
from jax.experimental import pallas as pl
from jax.experimental.pallas import tpu as pltpu
from jax.experimental.pallas import tpu_sc as plsc

F32 = jnp.float32

NC = 2
NS = 16
NW = NC * NS
CH = 125
RS = 80
BLK = 1024


def _mesh():
    return plsc.VectorSubcoreMesh(
        core_axis_name="c", subcore_axis_name="s", num_cores=NC, num_subcores=NS
    )


def _make_deg_kernel(NP, n_rows):
    rows_per_worker = n_rows // NW
    n_stages = rows_per_worker // RS
    rpt = NP // NS

    @functools.partial(
        pl.kernel,
        out_type=jax.ShapeDtypeStruct((NC * NP,), F32),
        mesh=_mesh(),
        scratch_types=[
            pltpu.VMEM((RS, CH), jnp.int32),
            pltpu.VMEM((128,), F32),
            pltpu.VMEM_SHARED((NP,), F32),
        ],
    )
    def deg_kernel(dst2d, zeros1, out, idx_v, ones_v, deg_sh):
        cid = lax.axis_index("c")
        sid = lax.axis_index("s")
        wid = cid * NS + sid
        base_row = wid * rows_per_worker
        r0 = sid * rpt
        pltpu.sync_copy(zeros1.at[pl.ds(r0, rpt)], deg_sh.at[pl.ds(r0, rpt)])
        for i in range(128 // 16):
            ones_v[pl.ds(i * 16, 16)] = jnp.ones((16,), F32)
        plsc.subcore_barrier()

        def stage(st, carry):
            pltpu.sync_copy(dst2d.at[pl.ds(base_row + st * RS, RS)], idx_v)

            def inner(j, c2):
                pltpu.sync_copy(ones_v.at[pl.ds(0, CH)],
                                deg_sh.at[idx_v.at[j]], add=True)
                return c2

            return lax.fori_loop(0, RS, inner, carry)

        lax.fori_loop(0, n_stages, stage, 0)
        plsc.subcore_barrier()
        pltpu.sync_copy(deg_sh.at[pl.ds(r0, rpt)],
                        out.at[pl.ds(cid * NP + r0, rpt)])

    return deg_kernel


def _make_time_agg_kernel(NP, T, n_rows):
    rows_per_worker = n_rows // NW
    n_stages = rows_per_worker // RS
    rpt = NP // NS

    @functools.partial(
        pl.kernel,
        out_type=jax.ShapeDtypeStruct((NC * T * NP,), F32),
        mesh=_mesh(),
        scratch_types=[
            pltpu.VMEM((RS, CH), jnp.int32),
            pltpu.VMEM((RS, CH), jnp.int32),
            pltpu.VMEM((4, T, CH), F32),
            pltpu.VMEM((T * CH,), F32),
            [pltpu.VMEM_SHARED((NP,), F32) for _ in range(T)],
            pltpu.SemaphoreType.DMA,
            pltpu.SemaphoreType.DMA,
        ],
    )
    def time_agg(*refs):
        xts = refs[0:T]
        src2d, dst2d, zeros1, out = refs[T:T + 4]
        src_v, dst_v, vals, drain_v, accs, gsem, ssem = refs[T + 4:]
        cid = lax.axis_index("c")
        sid = lax.axis_index("s")
        wid = cid * NS + sid
        base_row = wid * rows_per_worker
        r0 = sid * rpt
        for t in range(T):
            pltpu.sync_copy(zeros1.at[pl.ds(r0, rpt)],
                            accs[t].at[pl.ds(r0, rpt)])
        plsc.subcore_barrier()

        row_bytes = 4 * CH

        def _drain(sem, nbytes):
            pltpu.make_async_copy(zeros1.at[pl.ds(0, nbytes // 4)],
                                  drain_v.at[pl.ds(0, nbytes // 4)],
                                  sem).wait()

        def stage(st, carry):
            row0 = base_row + st * RS
            pltpu.sync_copy(src2d.at[pl.ds(row0, RS)], src_v)
            pltpu.sync_copy(dst2d.at[pl.ds(row0, RS)], dst_v)

            for t in range(T):
                pltpu.async_copy(xts[t].at[src_v.at[0]], vals.at[0, t], gsem)

            def inner(j, c2):
                p = lax.rem(j, 4)

                @pl.when(j >= 3)
                def _():
                    _drain(ssem, T * row_bytes)

                @pl.when(j < RS - 1)
                def _():
                    for t in range(T):
                        pltpu.async_copy(xts[t].at[src_v.at[j + 1]],
                                         vals.at[lax.rem(j + 1, 4), t], gsem)

                _drain(gsem, T * row_bytes)
                for t in range(T):
                    pltpu.async_copy(vals.at[p, t], accs[t].at[dst_v.at[j]],
                                     ssem, add=True)
                return c2

            r = lax.fori_loop(0, RS, inner, carry)
            _drain(ssem, 3 * T * row_bytes)
            return r

        lax.fori_loop(0, n_stages, stage, 0)
        plsc.subcore_barrier()
        for t in range(T):
            pltpu.sync_copy(accs[t].at[pl.ds(r0, rpt)],
                            out.at[pl.ds((cid * T + t) * NP + r0, rpt)])

    return time_agg


def _make_scalar_agg_kernel(NP, n_rows):
    rows_per_worker = n_rows // NW
    n_stages = rows_per_worker // RS
    rpt = NP // NS

    @functools.partial(
        pl.kernel,
        out_type=jax.ShapeDtypeStruct((NC * NP,), F32),
        mesh=_mesh(),
        scratch_types=[
            pltpu.VMEM((RS, CH), jnp.int32),
            pltpu.VMEM((RS, CH), jnp.int32),
            pltpu.VMEM((2, CH), F32),
            pltpu.VMEM((CH,), F32),
            pltpu.VMEM_SHARED((NP,), F32),
            pltpu.SemaphoreType.DMA,
            pltpu.SemaphoreType.DMA,
        ],
    )
    def scal_agg(ys, src2d, dst2d, zeros1, out,
                 src_v, dst_v, vals, drain_v, b_sh, gsem, ssem):
        cid = lax.axis_index("c")
        sid = lax.axis_index("s")
        wid = cid * NS + sid
        base_row = wid * rows_per_worker
        r0 = sid * rpt
        pltpu.sync_copy(zeros1.at[pl.ds(r0, rpt)], b_sh.at[pl.ds(r0, rpt)])
        plsc.subcore_barrier()

        row_bytes = 4 * CH

        def _drain(sem, nbytes):
            pltpu.make_async_copy(zeros1.at[pl.ds(0, nbytes // 4)],
                                  drain_v.at[pl.ds(0, nbytes // 4)],
                                  sem).wait()

        def stage(st, carry):
            row0 = base_row + st * RS
            pltpu.sync_copy(src2d.at[pl.ds(row0, RS)], src_v)
            pltpu.sync_copy(dst2d.at[pl.ds(row0, RS)], dst_v)
            pltpu.async_copy(ys.at[src_v.at[0]], vals.at[0], gsem)

            def inner(j, c2):
                p = lax.rem(j, 2)

                @pl.when(j < RS - 1)
                def _():
                    pltpu.async_copy(ys.at[src_v.at[j + 1]],
                                     vals.at[1 - p], gsem)

                @pl.when(j > 0)
                def _():
                    _drain(ssem, row_bytes)

                _drain(gsem, row_bytes)
                pltpu.async_copy(vals.at[p], b_sh.at[dst_v.at[j]],
                                 ssem, add=True)
                return c2

            r = lax.fori_loop(0, RS, inner, carry)
            _drain(ssem, row_bytes)
            return r

        lax.fori_loop(0, n_stages, stage, 0)
        plsc.subcore_barrier()
        pltpu.sync_copy(b_sh.at[pl.ds(r0, rpt)],
                        out.at[pl.ds(cid * NP + r0, rpt)])

    return scal_agg


def _prep_body(d, xT, dinv_o, xs_o):
    deg = d[0:1, :] + d[1:2, :] + 1.0
    dinv = lax.rsqrt(deg)
    dinv_o[...] = dinv
    xs_o[...] = dinv * xT[...]


def _make_gru_body(T, H):
    def _gru_body(aT, xsT, dinv, w1c, b1c, wih, whh, bic, bhc, w2r, ys_o):
        dv = dinv[...]
        s = dv * (aT[0:T, :] + aT[T:2 * T, :] + xsT[...])
        h = jnp.zeros((H, s.shape[1]), F32)
        for t in range(T):
            g = jax.nn.relu(w1c[...] * s[t:t + 1, :] + b1c[...])
            ui = jnp.dot(wih[...], g, preferred_element_type=F32) + bic[...]
            uh = jnp.dot(whh[...], h, preferred_element_type=F32) + bhc[...]
            r = jax.nn.sigmoid(ui[0:H] + uh[0:H])
            z = jax.nn.sigmoid(ui[H:2 * H] + uh[H:2 * H])
            nt = jnp.tanh(ui[2 * H:3 * H] + r * uh[2 * H:3 * H])
            h = (1.0 - z) * nt + z * h
        y = jnp.dot(w2r[...], h, preferred_element_type=F32)
        ys_o[...] = dv * y
    return _gru_body


def _out_body(b, ysc, dinv, b2s, o):
    o[...] = dinv[...] * (b[0:1, :] + b[1:2, :] + ysc[...]) + b2s[...]


def _row_spec(r):
    return pl.BlockSpec((r, BLK), lambda i: (0, i))


def _full_spec(shape):
    return pl.BlockSpec(shape, lambda i: tuple(0 for _ in shape))


def kernel(x_seq, edge_index, W1, b1, Wih, Whh, bih, bhh, W2, b2):
    N, T = x_seq.shape
    E = edge_index.shape[1]
    H = Whh.shape[1]
    NP = ((N + 2047) // 2048) * 2048
    nblocks = NP // BLK
    n_rows = E // CH

    xT = jnp.pad(x_seq.T, ((0, 0), (0, NP - N)))
    src2d = edge_index[0].reshape(n_rows, CH)
    dst2d = edge_index[1].reshape(n_rows, CH)
    zeros1 = jnp.zeros((NP,), F32)
    w1c = W1.reshape(H, 1)
    b1c = b1.reshape(H, 1)
    bic = bih.reshape(3 * H, 1)
    bhc = bhh.reshape(3 * H, 1)
    w2r = W2.reshape(1, H)
    b2s = b2.reshape(1, 1)

    deg2 = _make_deg_kernel(NP, n_rows)(dst2d, zeros1).reshape(NC, NP)

    dinv, xsT = pl.pallas_call(
        _prep_body,
        grid=(nblocks,),
        in_specs=[_row_spec(NC), _row_spec(T)],
        out_specs=[_row_spec(1), _row_spec(T)],
        out_shape=[jax.ShapeDtypeStruct((1, NP), F32),
                   jax.ShapeDtypeStruct((T, NP), F32)],
    )(deg2, xT)

    xts = [xsT[t] for t in range(T)]
    aT = _make_time_agg_kernel(NP, T, n_rows)(
        *xts, src2d, dst2d, zeros1).reshape(NC * T, NP)

    ysc = pl.pallas_call(
        _make_gru_body(T, H),
        grid=(nblocks,),
        in_specs=[_row_spec(NC * T), _row_spec(T), _row_spec(1),
                  _full_spec((H, 1)), _full_spec((H, 1)),
                  _full_spec((3 * H, H)), _full_spec((3 * H, H)),
                  _full_spec((3 * H, 1)), _full_spec((3 * H, 1)),
                  _full_spec((1, H))],
        out_specs=_row_spec(1),
        out_shape=jax.ShapeDtypeStruct((1, NP), F32),
    )(aT, xsT, dinv, w1c, b1c, Wih, Whh, bic, bhc, w2r)

    b2p = _make_scalar_agg_kernel(NP, n_rows)(
        ysc.reshape(NP), src2d, dst2d, zeros1).reshape(NC, NP)

    o = pl.pallas_call(
        _out_body,
        grid=(nblocks,),
        in_specs=[_row_spec(NC), _row_spec(1), _row_spec(1),
                  _full_spec((1, 1))],
        out_specs=_row_spec(1),
        out_shape=jax.ShapeDtypeStruct((1, NP), F32),
    )(b2p, ysc, dinv, b2s)

    return o[:, :N]


# speedup vs baseline: 118.2214x; 1.0710x over previous
import functools

import jax
import jax.numpy as jnp
from jax import lax
from jax.experimental import pallas as pl
from jax.experimental.pallas import tpu as pltpu
from jax.experimental.pallas import tpu_sc as plsc

F32 = jnp.float32

NC = 2
NS = 16
NW = NC * NS
CH = 125
RS = 80
BLK = 1024


def _mesh():
    return plsc.VectorSubcoreMesh(
        core_axis_name="c", subcore_axis_name="s", num_cores=NC, num_subcores=NS
    )


def _make_deg_kernel(NP, n_rows):
    rows_per_worker = n_rows // NW
    n_stages = rows_per_worker // RS
    rpt = NP // NS

    @functools.partial(
        pl.kernel,
        out_type=jax.ShapeDtypeStruct((NC * NP,), F32),
        mesh=_mesh(),
        scratch_types=[
            pltpu.VMEM((RS, CH), jnp.int32),
            pltpu.VMEM((128,), F32),
            pltpu.VMEM_SHARED((NP,), F32),
        ],
    )
    def deg_kernel(dst2d, zeros1, out, idx_v, ones_v, deg_sh):
        cid = lax.axis_index("c")
        sid = lax.axis_index("s")
        wid = cid * NS + sid
        base_row = wid * rows_per_worker
        r0 = sid * rpt
        pltpu.sync_copy(zeros1.at[pl.ds(r0, rpt)], deg_sh.at[pl.ds(r0, rpt)])
        for i in range(128 // 16):
            ones_v[pl.ds(i * 16, 16)] = jnp.ones((16,), F32)
        plsc.subcore_barrier()

        def stage(st, carry):
            pltpu.sync_copy(dst2d.at[pl.ds(base_row + st * RS, RS)], idx_v)

            def inner(j, c2):
                pltpu.sync_copy(ones_v.at[pl.ds(0, CH)],
                                deg_sh.at[idx_v.at[j]], add=True)
                return c2

            return lax.fori_loop(0, RS, inner, carry)

        lax.fori_loop(0, n_stages, stage, 0)
        plsc.subcore_barrier()
        pltpu.sync_copy(deg_sh.at[pl.ds(r0, rpt)],
                        out.at[pl.ds(cid * NP + r0, rpt)])

    return deg_kernel


def _make_time_agg_kernel(NP, T, n_rows):
    rows_per_worker = n_rows // NW
    n_stages = rows_per_worker // RS
    rpt = NP // NS

    @functools.partial(
        pl.kernel,
        out_type=jax.ShapeDtypeStruct((NC * T * NP,), F32),
        mesh=_mesh(),
        scratch_types=[
            pltpu.VMEM((RS, CH), jnp.int32),
            pltpu.VMEM((RS, CH), jnp.int32),
            pltpu.VMEM((4, T, CH), F32),
            pltpu.VMEM((T * CH,), F32),
            [pltpu.VMEM_SHARED((NP,), F32) for _ in range(T)],
            pltpu.SemaphoreType.DMA,
            pltpu.SemaphoreType.DMA,
        ],
    )
    def time_agg(*refs):
        xts = refs[0:T]
        src2d, dst2d, zeros1, out = refs[T:T + 4]
        src_v, dst_v, vals, drain_v, accs, gsem, ssem = refs[T + 4:]
        cid = lax.axis_index("c")
        sid = lax.axis_index("s")
        wid = cid * NS + sid
        base_row = wid * rows_per_worker
        r0 = sid * rpt
        for t in range(T):
            pltpu.sync_copy(zeros1.at[pl.ds(r0, rpt)],
                            accs[t].at[pl.ds(r0, rpt)])
        plsc.subcore_barrier()

        row_bytes = 4 * CH

        def _drain(sem, nbytes):
            pltpu.make_async_copy(zeros1.at[pl.ds(0, nbytes // 4)],
                                  drain_v.at[pl.ds(0, nbytes // 4)],
                                  sem).wait()

        def stage(st, carry):
            row0 = base_row + st * RS
            pltpu.sync_copy(src2d.at[pl.ds(row0, RS)], src_v)
            pltpu.sync_copy(dst2d.at[pl.ds(row0, RS)], dst_v)

            for t in range(T):
                pltpu.async_copy(xts[t].at[src_v.at[0]], vals.at[0, t], gsem)

            def inner(j, c2):
                p = lax.rem(j, 4)

                @pl.when(j >= 3)
                def _():
                    _drain(ssem, T * row_bytes)

                @pl.when(j < RS - 1)
                def _():
                    for t in range(T):
                        pltpu.async_copy(xts[t].at[src_v.at[j + 1]],
                                         vals.at[lax.rem(j + 1, 4), t], gsem)

                _drain(gsem, T * row_bytes)
                for t in range(T):
                    pltpu.async_copy(vals.at[p, t], accs[t].at[dst_v.at[j]],
                                     ssem, add=True)
                return c2

            r = lax.fori_loop(0, RS, inner, carry)
            _drain(ssem, 3 * T * row_bytes)
            return r

        lax.fori_loop(0, n_stages, stage, 0)
        plsc.subcore_barrier()
        for t in range(T):
            pltpu.sync_copy(accs[t].at[pl.ds(r0, rpt)],
                            out.at[pl.ds((cid * T + t) * NP + r0, rpt)])

    return time_agg


def _make_scalar_agg_kernel(NP, n_rows):
    rows_per_worker = n_rows // NW
    n_stages = rows_per_worker // RS
    rpt = NP // NS

    @functools.partial(
        pl.kernel,
        out_type=jax.ShapeDtypeStruct((NC * NP,), F32),
        mesh=_mesh(),
        scratch_types=[
            pltpu.VMEM((RS, CH), jnp.int32),
            pltpu.VMEM((RS, CH), jnp.int32),
            pltpu.VMEM((8, CH), F32),
            pltpu.VMEM((CH,), F32),
            pltpu.VMEM_SHARED((NP,), F32),
            pltpu.SemaphoreType.DMA,
            pltpu.SemaphoreType.DMA,
        ],
    )
    def scal_agg(ys, src2d, dst2d, zeros1, out,
                 src_v, dst_v, vals, drain_v, b_sh, gsem, ssem):
        cid = lax.axis_index("c")
        sid = lax.axis_index("s")
        wid = cid * NS + sid
        base_row = wid * rows_per_worker
        r0 = sid * rpt
        pltpu.sync_copy(zeros1.at[pl.ds(r0, rpt)], b_sh.at[pl.ds(r0, rpt)])
        plsc.subcore_barrier()

        row_bytes = 4 * CH

        def _drain(sem, nbytes):
            pltpu.make_async_copy(zeros1.at[pl.ds(0, nbytes // 4)],
                                  drain_v.at[pl.ds(0, nbytes // 4)],
                                  sem).wait()

        D = 8

        def stage(st, carry):
            row0 = base_row + st * RS
            pltpu.sync_copy(src2d.at[pl.ds(row0, RS)], src_v)
            pltpu.sync_copy(dst2d.at[pl.ds(row0, RS)], dst_v)
            for k in range(D - 1):
                pltpu.async_copy(ys.at[src_v.at[k]], vals.at[k], gsem)

            def inner(j, c2):
                @pl.when(j > 0)
                def _():
                    _drain(ssem, row_bytes)

                @pl.when(j + D - 1 < RS)
                def _():
                    pltpu.async_copy(ys.at[src_v.at[j + D - 1]],
                                     vals.at[lax.rem(j + D - 1, D)], gsem)

                _drain(gsem, row_bytes)
                pltpu.async_copy(vals.at[lax.rem(j, D)],
                                 b_sh.at[dst_v.at[j]], ssem, add=True)
                return c2

            r = lax.fori_loop(0, RS, inner, carry)
            _drain(ssem, row_bytes)
            return r

        lax.fori_loop(0, n_stages, stage, 0)
        plsc.subcore_barrier()
        pltpu.sync_copy(b_sh.at[pl.ds(r0, rpt)],
                        out.at[pl.ds(cid * NP + r0, rpt)])

    return scal_agg


def _prep_body(d, xT, dinv_o, xs_o):
    deg = d[0:1, :] + d[1:2, :] + 1.0
    dinv = lax.rsqrt(deg)
    dinv_o[...] = dinv
    xs_o[...] = dinv * xT[...]


def _make_gru_body(T, H):
    def _gru_body(aT, xsT, dinv, w1c, b1c, wih, whh, bic, bhc, w2r, ys_o):
        dv = dinv[...]
        s = dv * (aT[0:T, :] + aT[T:2 * T, :] + xsT[...])
        h = jnp.zeros((H, s.shape[1]), F32)
        for t in range(T):
            g = jax.nn.relu(w1c[...] * s[t:t + 1, :] + b1c[...])
            ui = jnp.dot(wih[...], g, preferred_element_type=F32) + bic[...]
            uh = jnp.dot(whh[...], h, preferred_element_type=F32) + bhc[...]
            r = jax.nn.sigmoid(ui[0:H] + uh[0:H])
            z = jax.nn.sigmoid(ui[H:2 * H] + uh[H:2 * H])
            nt = jnp.tanh(ui[2 * H:3 * H] + r * uh[2 * H:3 * H])
            h = (1.0 - z) * nt + z * h
        y = jnp.dot(w2r[...], h, preferred_element_type=F32)
        ys_o[...] = dv * y
    return _gru_body


def _out_body(b, ysc, dinv, b2s, o):
    o[...] = dinv[...] * (b[0:1, :] + b[1:2, :] + ysc[...]) + b2s[...]


def _row_spec(r):
    return pl.BlockSpec((r, BLK), lambda i: (0, i))


def _full_spec(shape):
    return pl.BlockSpec(shape, lambda i: tuple(0 for _ in shape))


def kernel(x_seq, edge_index, W1, b1, Wih, Whh, bih, bhh, W2, b2):
    N, T = x_seq.shape
    E = edge_index.shape[1]
    H = Whh.shape[1]
    NP = ((N + 2047) // 2048) * 2048
    nblocks = NP // BLK
    n_rows = E // CH

    xT = jnp.pad(x_seq.T, ((0, 0), (0, NP - N)))
    src2d = edge_index[0].reshape(n_rows, CH)
    dst2d = edge_index[1].reshape(n_rows, CH)
    zeros1 = jnp.zeros((NP,), F32)
    w1c = W1.reshape(H, 1)
    b1c = b1.reshape(H, 1)
    bic = bih.reshape(3 * H, 1)
    bhc = bhh.reshape(3 * H, 1)
    w2r = W2.reshape(1, H)
    b2s = b2.reshape(1, 1)

    deg2 = _make_deg_kernel(NP, n_rows)(dst2d, zeros1).reshape(NC, NP)

    dinv, xsT = pl.pallas_call(
        _prep_body,
        grid=(nblocks,),
        in_specs=[_row_spec(NC), _row_spec(T)],
        out_specs=[_row_spec(1), _row_spec(T)],
        out_shape=[jax.ShapeDtypeStruct((1, NP), F32),
                   jax.ShapeDtypeStruct((T, NP), F32)],
    )(deg2, xT)

    xts = [xsT[t] for t in range(T)]
    aT = _make_time_agg_kernel(NP, T, n_rows)(
        *xts, src2d, dst2d, zeros1).reshape(NC * T, NP)

    ysc = pl.pallas_call(
        _make_gru_body(T, H),
        grid=(nblocks,),
        in_specs=[_row_spec(NC * T), _row_spec(T), _row_spec(1),
                  _full_spec((H, 1)), _full_spec((H, 1)),
                  _full_spec((3 * H, H)), _full_spec((3 * H, H)),
                  _full_spec((3 * H, 1)), _full_spec((3 * H, 1)),
                  _full_spec((1, H))],
        out_specs=_row_spec(1),
        out_shape=jax.ShapeDtypeStruct((1, NP), F32),
    )(aT, xsT, dinv, w1c, b1c, Wih, Whh, bic, bhc, w2r)

    b2p = _make_scalar_agg_kernel(NP, n_rows)(
        ysc.reshape(NP), src2d, dst2d, zeros1).reshape(NC, NP)

    o = pl.pallas_call(
        _out_body,
        grid=(nblocks,),
        in_specs=[_row_spec(NC), _row_spec(1), _row_spec(1),
                  _full_spec((1, 1))],
        out_specs=_row_spec(1),
        out_shape=jax.ShapeDtypeStruct((1, NP), F32),
    )(b2p, ysc, dinv, b2s)

    return o[:, :N]


# speedup vs baseline: 123.2740x; 1.0427x over previous
import functools

import jax
import jax.numpy as jnp
from jax import lax
from jax.experimental import pallas as pl
from jax.experimental.pallas import tpu as pltpu
from jax.experimental.pallas import tpu_sc as plsc

F32 = jnp.float32

NC = 2
NS = 16
NW = NC * NS
CH = 125
RS = 80
BLK = 1024


def _mesh():
    return plsc.VectorSubcoreMesh(
        core_axis_name="c", subcore_axis_name="s", num_cores=NC, num_subcores=NS
    )


def _make_deg_kernel(NP, n_rows):
    rows_per_worker = n_rows // NW
    n_stages = rows_per_worker // RS
    rpt = NP // NS

    @functools.partial(
        pl.kernel,
        out_type=jax.ShapeDtypeStruct((NC * NP,), F32),
        mesh=_mesh(),
        scratch_types=[
            pltpu.VMEM((RS, CH), jnp.int32),
            pltpu.VMEM((128,), F32),
            pltpu.VMEM_SHARED((NP,), F32),
        ],
    )
    def deg_kernel(dst2d, zeros1, out, idx_v, ones_v, deg_sh):
        cid = lax.axis_index("c")
        sid = lax.axis_index("s")
        wid = cid * NS + sid
        base_row = wid * rows_per_worker
        r0 = sid * rpt
        pltpu.sync_copy(zeros1.at[pl.ds(r0, rpt)], deg_sh.at[pl.ds(r0, rpt)])
        for i in range(128 // 16):
            ones_v[pl.ds(i * 16, 16)] = jnp.ones((16,), F32)
        plsc.subcore_barrier()

        def stage(st, carry):
            pltpu.sync_copy(dst2d.at[pl.ds(base_row + st * RS, RS)], idx_v)

            def inner(j, c2):
                pltpu.sync_copy(ones_v.at[pl.ds(0, CH)],
                                deg_sh.at[idx_v.at[j]], add=True)
                return c2

            return lax.fori_loop(0, RS, inner, carry)

        lax.fori_loop(0, n_stages, stage, 0)
        plsc.subcore_barrier()
        pltpu.sync_copy(deg_sh.at[pl.ds(r0, rpt)],
                        out.at[pl.ds(cid * NP + r0, rpt)])

    return deg_kernel


def _make_time_agg_kernel(NP, T, n_rows):
    rows_per_worker = n_rows // NW
    n_stages = rows_per_worker // RS
    rpt = NP // NS

    @functools.partial(
        pl.kernel,
        out_type=jax.ShapeDtypeStruct((NC * T * NP,), F32),
        mesh=_mesh(),
        scratch_types=[
            pltpu.VMEM((RS, CH), jnp.int32),
            pltpu.VMEM((RS, CH), jnp.int32),
            pltpu.VMEM((4, T, CH), F32),
            pltpu.VMEM((T * CH,), F32),
            [pltpu.VMEM_SHARED((NP,), F32) for _ in range(T)],
            pltpu.SemaphoreType.DMA,
            pltpu.SemaphoreType.DMA,
        ],
    )
    def time_agg(*refs):
        xts = refs[0:T]
        src2d, dst2d, zeros1, out = refs[T:T + 4]
        src_v, dst_v, vals, drain_v, accs, gsem, ssem = refs[T + 4:]
        cid = lax.axis_index("c")
        sid = lax.axis_index("s")
        wid = cid * NS + sid
        base_row = wid * rows_per_worker
        r0 = sid * rpt
        for t in range(T):
            pltpu.sync_copy(zeros1.at[pl.ds(r0, rpt)],
                            accs[t].at[pl.ds(r0, rpt)])
        plsc.subcore_barrier()

        row_bytes = 4 * CH

        def _drain(sem, nbytes):
            pltpu.make_async_copy(zeros1.at[pl.ds(0, nbytes // 4)],
                                  drain_v.at[pl.ds(0, nbytes // 4)],
                                  sem).wait()

        D = 4

        def stage(st, carry):
            row0 = base_row + st * RS
            pltpu.sync_copy(src2d.at[pl.ds(row0, RS)], src_v)
            pltpu.sync_copy(dst2d.at[pl.ds(row0, RS)], dst_v)

            for k in range(D - 1):
                for t in range(T):
                    pltpu.async_copy(xts[t].at[src_v.at[k]], vals.at[k, t],
                                     gsem)

            def inner(j, c2):
                @pl.when(j > 0)
                def _():
                    _drain(ssem, T * row_bytes)

                @pl.when(j + D - 1 < RS)
                def _():
                    for t in range(T):
                        pltpu.async_copy(xts[t].at[src_v.at[j + D - 1]],
                                         vals.at[lax.rem(j + D - 1, D), t],
                                         gsem)

                _drain(gsem, T * row_bytes)
                for t in range(T):
                    pltpu.async_copy(vals.at[lax.rem(j, D), t],
                                     accs[t].at[dst_v.at[j]], ssem, add=True)
                return c2

            r = lax.fori_loop(0, RS, inner, carry)
            _drain(ssem, T * row_bytes)
            return r

        lax.fori_loop(0, n_stages, stage, 0)
        plsc.subcore_barrier()
        for t in range(T):
            pltpu.sync_copy(accs[t].at[pl.ds(r0, rpt)],
                            out.at[pl.ds((cid * T + t) * NP + r0, rpt)])

    return time_agg


def _make_scalar_agg_kernel(NP, n_rows):
    rows_per_worker = n_rows // NW
    n_stages = rows_per_worker // RS
    rpt = NP // NS

    @functools.partial(
        pl.kernel,
        out_type=jax.ShapeDtypeStruct((NC * NP,), F32),
        mesh=_mesh(),
        scratch_types=[
            pltpu.VMEM((RS, CH), jnp.int32),
            pltpu.VMEM((RS, CH), jnp.int32),
            pltpu.VMEM((8, CH), F32),
            pltpu.VMEM((CH,), F32),
            pltpu.VMEM_SHARED((NP,), F32),
            pltpu.SemaphoreType.DMA,
            pltpu.SemaphoreType.DMA,
        ],
    )
    def scal_agg(ys, src2d, dst2d, zeros1, out,
                 src_v, dst_v, vals, drain_v, b_sh, gsem, ssem):
        cid = lax.axis_index("c")
        sid = lax.axis_index("s")
        wid = cid * NS + sid
        base_row = wid * rows_per_worker
        r0 = sid * rpt
        pltpu.sync_copy(zeros1.at[pl.ds(r0, rpt)], b_sh.at[pl.ds(r0, rpt)])
        plsc.subcore_barrier()

        row_bytes = 4 * CH

        def _drain(sem, nbytes):
            pltpu.make_async_copy(zeros1.at[pl.ds(0, nbytes // 4)],
                                  drain_v.at[pl.ds(0, nbytes // 4)],
                                  sem).wait()

        D = 8

        def stage(st, carry):
            row0 = base_row + st * RS
            pltpu.sync_copy(src2d.at[pl.ds(row0, RS)], src_v)
            pltpu.sync_copy(dst2d.at[pl.ds(row0, RS)], dst_v)
            for k in range(D - 1):
                pltpu.async_copy(ys.at[src_v.at[k]], vals.at[k], gsem)

            def inner(j, c2):
                @pl.when(j > 0)
                def _():
                    _drain(ssem, row_bytes)

                @pl.when(j + D - 1 < RS)
                def _():
                    pltpu.async_copy(ys.at[src_v.at[j + D - 1]],
                                     vals.at[lax.rem(j + D - 1, D)], gsem)

                _drain(gsem, row_bytes)
                pltpu.async_copy(vals.at[lax.rem(j, D)],
                                 b_sh.at[dst_v.at[j]], ssem, add=True)
                return c2

            r = lax.fori_loop(0, RS, inner, carry)
            _drain(ssem, row_bytes)
            return r

        lax.fori_loop(0, n_stages, stage, 0)
        plsc.subcore_barrier()
        pltpu.sync_copy(b_sh.at[pl.ds(r0, rpt)],
                        out.at[pl.ds(cid * NP + r0, rpt)])

    return scal_agg


def _prep_body(d, xT, dinv_o, xs_o):
    deg = d[0:1, :] + d[1:2, :] + 1.0
    dinv = lax.rsqrt(deg)
    dinv_o[...] = dinv
    xs_o[...] = dinv * xT[...]


def _make_gru_body(T, H):
    def _gru_body(aT, xsT, dinv, w1c, b1c, wih, whh, bic, bhc, w2r, ys_o):
        dv = dinv[...]
        s = dv * (aT[0:T, :] + aT[T:2 * T, :] + xsT[...])
        h = jnp.zeros((H, s.shape[1]), F32)
        for t in range(T):
            g = jax.nn.relu(w1c[...] * s[t:t + 1, :] + b1c[...])
            ui = jnp.dot(wih[...], g, preferred_element_type=F32) + bic[...]
            uh = jnp.dot(whh[...], h, preferred_element_type=F32) + bhc[...]
            r = jax.nn.sigmoid(ui[0:H] + uh[0:H])
            z = jax.nn.sigmoid(ui[H:2 * H] + uh[H:2 * H])
            nt = jnp.tanh(ui[2 * H:3 * H] + r * uh[2 * H:3 * H])
            h = (1.0 - z) * nt + z * h
        y = jnp.dot(w2r[...], h, preferred_element_type=F32)
        ys_o[...] = dv * y
    return _gru_body


def _out_body(b, ysc, dinv, b2s, o):
    o[...] = dinv[...] * (b[0:1, :] + b[1:2, :] + ysc[...]) + b2s[...]


def _row_spec(r):
    return pl.BlockSpec((r, BLK), lambda i: (0, i))


def _full_spec(shape):
    return pl.BlockSpec(shape, lambda i: tuple(0 for _ in shape))


def kernel(x_seq, edge_index, W1, b1, Wih, Whh, bih, bhh, W2, b2):
    N, T = x_seq.shape
    E = edge_index.shape[1]
    H = Whh.shape[1]
    NP = ((N + 2047) // 2048) * 2048
    nblocks = NP // BLK
    n_rows = E // CH

    xT = jnp.pad(x_seq.T, ((0, 0), (0, NP - N)))
    src2d = edge_index[0].reshape(n_rows, CH)
    dst2d = edge_index[1].reshape(n_rows, CH)
    zeros1 = jnp.zeros((NP,), F32)
    w1c = W1.reshape(H, 1)
    b1c = b1.reshape(H, 1)
    bic = bih.reshape(3 * H, 1)
    bhc = bhh.reshape(3 * H, 1)
    w2r = W2.reshape(1, H)
    b2s = b2.reshape(1, 1)

    deg2 = _make_deg_kernel(NP, n_rows)(dst2d, zeros1).reshape(NC, NP)

    dinv, xsT = pl.pallas_call(
        _prep_body,
        grid=(nblocks,),
        in_specs=[_row_spec(NC), _row_spec(T)],
        out_specs=[_row_spec(1), _row_spec(T)],
        out_shape=[jax.ShapeDtypeStruct((1, NP), F32),
                   jax.ShapeDtypeStruct((T, NP), F32)],
    )(deg2, xT)

    xts = [xsT[t] for t in range(T)]
    aT = _make_time_agg_kernel(NP, T, n_rows)(
        *xts, src2d, dst2d, zeros1).reshape(NC * T, NP)

    ysc = pl.pallas_call(
        _make_gru_body(T, H),
        grid=(nblocks,),
        in_specs=[_row_spec(NC * T), _row_spec(T), _row_spec(1),
                  _full_spec((H, 1)), _full_spec((H, 1)),
                  _full_spec((3 * H, H)), _full_spec((3 * H, H)),
                  _full_spec((3 * H, 1)), _full_spec((3 * H, 1)),
                  _full_spec((1, H))],
        out_specs=_row_spec(1),
        out_shape=jax.ShapeDtypeStruct((1, NP), F32),
    )(aT, xsT, dinv, w1c, b1c, Wih, Whh, bic, bhc, w2r)

    b2p = _make_scalar_agg_kernel(NP, n_rows)(
        ysc.reshape(NP), src2d, dst2d, zeros1).reshape(NC, NP)

    o = pl.pallas_call(
        _out_body,
        grid=(nblocks,),
        in_specs=[_row_spec(NC), _row_spec(1), _row_spec(1),
                  _full_spec((1, 1))],
        out_specs=_row_spec(1),
        out_shape=jax.ShapeDtypeStruct((1, NP), F32),
    )(b2p, ysc, dinv, b2s)

    return o[:, :N]


# speedup vs baseline: 131.2830x; 1.0650x over previous
import functools

import jax
import jax.numpy as jnp
from jax import lax
from jax.experimental import pallas as pl
from jax.experimental.pallas import tpu as pltpu
from jax.experimental.pallas import tpu_sc as plsc

F32 = jnp.float32

NC = 2
NS = 16
NW = NC * NS
CH = 125
RS = 80
BLK = 1024


def _mesh():
    return plsc.VectorSubcoreMesh(
        core_axis_name="c", subcore_axis_name="s", num_cores=NC, num_subcores=NS
    )


def _make_deg_kernel(NP, n_rows):
    rows_per_worker = n_rows // NW
    n_stages = rows_per_worker // RS
    rpt = NP // NS

    @functools.partial(
        pl.kernel,
        out_type=jax.ShapeDtypeStruct((NC * NP,), F32),
        mesh=_mesh(),
        scratch_types=[
            pltpu.VMEM((RS, CH), jnp.int32),
            pltpu.VMEM((128,), F32),
            pltpu.VMEM((CH,), F32),
            pltpu.VMEM_SHARED((NP,), F32),
            pltpu.SemaphoreType.DMA,
        ],
    )
    def deg_kernel(dst2d, zeros1, out, idx_v, ones_v, drain_v, deg_sh, ssem):
        cid = lax.axis_index("c")
        sid = lax.axis_index("s")
        wid = cid * NS + sid
        base_row = wid * rows_per_worker
        r0 = sid * rpt
        pltpu.sync_copy(zeros1.at[pl.ds(r0, rpt)], deg_sh.at[pl.ds(r0, rpt)])
        for i in range(128 // 16):
            ones_v[pl.ds(i * 16, 16)] = jnp.ones((16,), F32)
        plsc.subcore_barrier()

        row_bytes = 4 * CH
        LAG = 8

        def _drain(nbytes):
            pltpu.make_async_copy(zeros1.at[pl.ds(0, nbytes // 4)],
                                  drain_v.at[pl.ds(0, nbytes // 4)],
                                  ssem).wait()

        def stage(st, carry):
            pltpu.sync_copy(dst2d.at[pl.ds(base_row + st * RS, RS)], idx_v)

            def inner(j, c2):
                @pl.when(j >= LAG)
                def _():
                    _drain(row_bytes)

                pltpu.async_copy(ones_v.at[pl.ds(0, CH)],
                                 deg_sh.at[idx_v.at[j]], ssem, add=True)
                return c2

            r = lax.fori_loop(0, RS, inner, carry)
            _drain(LAG * row_bytes)
            return r

        lax.fori_loop(0, n_stages, stage, 0)
        plsc.subcore_barrier()
        pltpu.sync_copy(deg_sh.at[pl.ds(r0, rpt)],
                        out.at[pl.ds(cid * NP + r0, rpt)])

    return deg_kernel


def _make_time_agg_kernel(NP, T, n_rows):
    rows_per_worker = n_rows // NW
    n_stages = rows_per_worker // RS
    rpt = NP // NS

    @functools.partial(
        pl.kernel,
        out_type=jax.ShapeDtypeStruct((NC * T * NP,), F32),
        mesh=_mesh(),
        scratch_types=[
            pltpu.VMEM((RS, CH), jnp.int32),
            pltpu.VMEM((RS, CH), jnp.int32),
            pltpu.VMEM((8, T, CH), F32),
            pltpu.VMEM((T * CH,), F32),
            [pltpu.VMEM_SHARED((NP,), F32) for _ in range(T)],
            pltpu.SemaphoreType.DMA,
            pltpu.SemaphoreType.DMA,
        ],
    )
    def time_agg(*refs):
        xts = refs[0:T]
        src2d, dst2d, zeros1, out = refs[T:T + 4]
        src_v, dst_v, vals, drain_v, accs, gsem, ssem = refs[T + 4:]
        cid = lax.axis_index("c")
        sid = lax.axis_index("s")
        wid = cid * NS + sid
        base_row = wid * rows_per_worker
        r0 = sid * rpt
        for t in range(T):
            pltpu.sync_copy(zeros1.at[pl.ds(r0, rpt)],
                            accs[t].at[pl.ds(r0, rpt)])
        plsc.subcore_barrier()

        row_bytes = 4 * CH

        def _drain(sem, nbytes):
            pltpu.make_async_copy(zeros1.at[pl.ds(0, nbytes // 4)],
                                  drain_v.at[pl.ds(0, nbytes // 4)],
                                  sem).wait()

        D = 8

        def stage(st, carry):
            row0 = base_row + st * RS
            pltpu.sync_copy(src2d.at[pl.ds(row0, RS)], src_v)
            pltpu.sync_copy(dst2d.at[pl.ds(row0, RS)], dst_v)

            for k in range(D - 1):
                for t in range(T):
                    pltpu.async_copy(xts[t].at[src_v.at[k]], vals.at[k, t],
                                     gsem)

            def inner(j, c2):
                @pl.when(j > 0)
                def _():
                    _drain(ssem, T * row_bytes)

                @pl.when(j + D - 1 < RS)
                def _():
                    for t in range(T):
                        pltpu.async_copy(xts[t].at[src_v.at[j + D - 1]],
                                         vals.at[lax.rem(j + D - 1, D), t],
                                         gsem)

                _drain(gsem, T * row_bytes)
                for t in range(T):
                    pltpu.async_copy(vals.at[lax.rem(j, D), t],
                                     accs[t].at[dst_v.at[j]], ssem, add=True)
                return c2

            r = lax.fori_loop(0, RS, inner, carry)
            _drain(ssem, T * row_bytes)
            return r

        lax.fori_loop(0, n_stages, stage, 0)
        plsc.subcore_barrier()
        for t in range(T):
            pltpu.sync_copy(accs[t].at[pl.ds(r0, rpt)],
                            out.at[pl.ds((cid * T + t) * NP + r0, rpt)])

    return time_agg


def _make_scalar_agg_kernel(NP, n_rows):
    rows_per_worker = n_rows // NW
    n_stages = rows_per_worker // RS
    rpt = NP // NS

    @functools.partial(
        pl.kernel,
        out_type=jax.ShapeDtypeStruct((NC * NP,), F32),
        mesh=_mesh(),
        scratch_types=[
            pltpu.VMEM((RS, CH), jnp.int32),
            pltpu.VMEM((RS, CH), jnp.int32),
            pltpu.VMEM((8, CH), F32),
            pltpu.VMEM((CH,), F32),
            pltpu.VMEM_SHARED((NP,), F32),
            pltpu.SemaphoreType.DMA,
            pltpu.SemaphoreType.DMA,
        ],
    )
    def scal_agg(ys, src2d, dst2d, zeros1, out,
                 src_v, dst_v, vals, drain_v, b_sh, gsem, ssem):
        cid = lax.axis_index("c")
        sid = lax.axis_index("s")
        wid = cid * NS + sid
        base_row = wid * rows_per_worker
        r0 = sid * rpt
        pltpu.sync_copy(zeros1.at[pl.ds(r0, rpt)], b_sh.at[pl.ds(r0, rpt)])
        plsc.subcore_barrier()

        row_bytes = 4 * CH

        def _drain(sem, nbytes):
            pltpu.make_async_copy(zeros1.at[pl.ds(0, nbytes // 4)],
                                  drain_v.at[pl.ds(0, nbytes // 4)],
                                  sem).wait()

        D = 8

        def stage(st, carry):
            row0 = base_row + st * RS
            pltpu.sync_copy(src2d.at[pl.ds(row0, RS)], src_v)
            pltpu.sync_copy(dst2d.at[pl.ds(row0, RS)], dst_v)
            for k in range(D - 1):
                pltpu.async_copy(ys.at[src_v.at[k]], vals.at[k], gsem)

            def inner(j, c2):
                @pl.when(j > 0)
                def _():
                    _drain(ssem, row_bytes)

                @pl.when(j + D - 1 < RS)
                def _():
                    pltpu.async_copy(ys.at[src_v.at[j + D - 1]],
                                     vals.at[lax.rem(j + D - 1, D)], gsem)

                _drain(gsem, row_bytes)
                pltpu.async_copy(vals.at[lax.rem(j, D)],
                                 b_sh.at[dst_v.at[j]], ssem, add=True)
                return c2

            r = lax.fori_loop(0, RS, inner, carry)
            _drain(ssem, row_bytes)
            return r

        lax.fori_loop(0, n_stages, stage, 0)
        plsc.subcore_barrier()
        pltpu.sync_copy(b_sh.at[pl.ds(r0, rpt)],
                        out.at[pl.ds(cid * NP + r0, rpt)])

    return scal_agg


def _prep_body(d, xT, dinv_o, xs_o):
    deg = d[0:1, :] + d[1:2, :] + 1.0
    dinv = lax.rsqrt(deg)
    dinv_o[...] = dinv
    xs_o[...] = dinv * xT[...]


def _make_gru_body(T, H):
    def _gru_body(aT, xsT, dinv, w1c, b1c, wih, whh, bic, bhc, w2r, ys_o):
        dv = dinv[...]
        s = dv * (aT[0:T, :] + aT[T:2 * T, :] + xsT[...])
        h = jnp.zeros((H, s.shape[1]), F32)
        for t in range(T):
            g = jax.nn.relu(w1c[...] * s[t:t + 1, :] + b1c[...])
            ui = jnp.dot(wih[...], g, preferred_element_type=F32) + bic[...]
            uh = jnp.dot(whh[...], h, preferred_element_type=F32) + bhc[...]
            r = jax.nn.sigmoid(ui[0:H] + uh[0:H])
            z = jax.nn.sigmoid(ui[H:2 * H] + uh[H:2 * H])
            nt = jnp.tanh(ui[2 * H:3 * H] + r * uh[2 * H:3 * H])
            h = (1.0 - z) * nt + z * h
        y = jnp.dot(w2r[...], h, preferred_element_type=F32)
        ys_o[...] = dv * y
    return _gru_body


def _out_body(b, ysc, dinv, b2s, o):
    o[...] = dinv[...] * (b[0:1, :] + b[1:2, :] + ysc[...]) + b2s[...]


def _row_spec(r):
    return pl.BlockSpec((r, BLK), lambda i: (0, i))


def _full_spec(shape):
    return pl.BlockSpec(shape, lambda i: tuple(0 for _ in shape))


def kernel(x_seq, edge_index, W1, b1, Wih, Whh, bih, bhh, W2, b2):
    N, T = x_seq.shape
    E = edge_index.shape[1]
    H = Whh.shape[1]
    NP = ((N + 2047) // 2048) * 2048
    nblocks = NP // BLK
    n_rows = E // CH

    xT = jnp.pad(x_seq.T, ((0, 0), (0, NP - N)))
    src2d = edge_index[0].reshape(n_rows, CH)
    dst2d = edge_index[1].reshape(n_rows, CH)
    zeros1 = jnp.zeros((NP,), F32)
    w1c = W1.reshape(H, 1)
    b1c = b1.reshape(H, 1)
    bic = bih.reshape(3 * H, 1)
    bhc = bhh.reshape(3 * H, 1)
    w2r = W2.reshape(1, H)
    b2s = b2.reshape(1, 1)

    deg2 = _make_deg_kernel(NP, n_rows)(dst2d, zeros1).reshape(NC, NP)

    dinv, xsT = pl.pallas_call(
        _prep_body,
        grid=(nblocks,),
        in_specs=[_row_spec(NC), _row_spec(T)],
        out_specs=[_row_spec(1), _row_spec(T)],
        out_shape=[jax.ShapeDtypeStruct((1, NP), F32),
                   jax.ShapeDtypeStruct((T, NP), F32)],
    )(deg2, xT)

    xts = [xsT[t] for t in range(T)]
    aT = _make_time_agg_kernel(NP, T, n_rows)(
        *xts, src2d, dst2d, zeros1).reshape(NC * T, NP)

    ysc = pl.pallas_call(
        _make_gru_body(T, H),
        grid=(nblocks,),
        in_specs=[_row_spec(NC * T), _row_spec(T), _row_spec(1),
                  _full_spec((H, 1)), _full_spec((H, 1)),
                  _full_spec((3 * H, H)), _full_spec((3 * H, H)),
                  _full_spec((3 * H, 1)), _full_spec((3 * H, 1)),
                  _full_spec((1, H))],
        out_specs=_row_spec(1),
        out_shape=jax.ShapeDtypeStruct((1, NP), F32),
    )(aT, xsT, dinv, w1c, b1c, Wih, Whh, bic, bhc, w2r)

    b2p = _make_scalar_agg_kernel(NP, n_rows)(
        ysc.reshape(NP), src2d, dst2d, zeros1).reshape(NC, NP)

    o = pl.pallas_call(
        _out_body,
        grid=(nblocks,),
        in_specs=[_row_spec(NC), _row_spec(1), _row_spec(1),
                  _full_spec((1, 1))],
        out_specs=_row_spec(1),
        out_shape=jax.ShapeDtypeStruct((1, NP), F32),
    )(b2p, ysc, dinv, b2s)

    return o[:, :N]


# speedup vs baseline: 143.5385x; 1.0934x over previous
import functools

import jax
import jax.numpy as jnp
from jax import lax
from jax.experimental import pallas as pl
from jax.experimental.pallas import tpu as pltpu
from jax.experimental.pallas import tpu_sc as plsc

F32 = jnp.float32

NC = 2
NS = 16
NW = NC * NS
CH = 125
RS = 80
BLK = 6400


def _mesh():
    return plsc.VectorSubcoreMesh(
        core_axis_name="c", subcore_axis_name="s", num_cores=NC, num_subcores=NS
    )


def _make_deg_kernel(NP, n_rows):
    rows_per_worker = n_rows // NW
    n_stages = rows_per_worker // RS
    rpt = NP // NS

    @functools.partial(
        pl.kernel,
        out_type=jax.ShapeDtypeStruct((NC * NP,), F32),
        mesh=_mesh(),
        scratch_types=[
            pltpu.VMEM((RS, CH), jnp.int32),
            pltpu.VMEM((128,), F32),
            pltpu.VMEM((CH,), F32),
            pltpu.VMEM_SHARED((NP,), F32),
            pltpu.SemaphoreType.DMA,
        ],
    )
    def deg_kernel(dst2d, zeros1, out, idx_v, ones_v, drain_v, deg_sh, ssem):
        cid = lax.axis_index("c")
        sid = lax.axis_index("s")
        wid = cid * NS + sid
        base_row = wid * rows_per_worker
        r0 = sid * rpt
        pltpu.sync_copy(zeros1.at[pl.ds(r0, rpt)], deg_sh.at[pl.ds(r0, rpt)])
        for i in range(128 // 16):
            ones_v[pl.ds(i * 16, 16)] = jnp.ones((16,), F32)
        plsc.subcore_barrier()

        row_bytes = 4 * CH
        LAG = 8

        def _drain(nbytes):
            pltpu.make_async_copy(zeros1.at[pl.ds(0, nbytes // 4)],
                                  drain_v.at[pl.ds(0, nbytes // 4)],
                                  ssem).wait()

        def stage(st, carry):
            pltpu.sync_copy(dst2d.at[pl.ds(base_row + st * RS, RS)], idx_v)

            def inner(j, c2):
                @pl.when(j >= LAG)
                def _():
                    _drain(row_bytes)

                pltpu.async_copy(ones_v.at[pl.ds(0, CH)],
                                 deg_sh.at[idx_v.at[j]], ssem, add=True)
                return c2

            r = lax.fori_loop(0, RS, inner, carry)
            _drain(LAG * row_bytes)
            return r

        lax.fori_loop(0, n_stages, stage, 0)
        plsc.subcore_barrier()
        pltpu.sync_copy(deg_sh.at[pl.ds(r0, rpt)],
                        out.at[pl.ds(cid * NP + r0, rpt)])

    return deg_kernel


def _make_time_agg_kernel(NP, T, n_rows):
    rows_per_worker = n_rows // NW
    n_stages = rows_per_worker // RS
    rpt = NP // NS

    @functools.partial(
        pl.kernel,
        out_type=jax.ShapeDtypeStruct((NC * T * NP,), F32),
        mesh=_mesh(),
        scratch_types=[
            pltpu.VMEM((RS, CH), jnp.int32),
            pltpu.VMEM((RS, CH), jnp.int32),
            pltpu.VMEM((8, T, CH), F32),
            pltpu.VMEM((T * CH,), F32),
            [pltpu.VMEM_SHARED((NP,), F32) for _ in range(T)],
            pltpu.SemaphoreType.DMA,
            pltpu.SemaphoreType.DMA,
        ],
    )
    def time_agg(*refs):
        xts = refs[0:T]
        src2d, dst2d, zeros1, out = refs[T:T + 4]
        src_v, dst_v, vals, drain_v, accs, gsem, ssem = refs[T + 4:]
        cid = lax.axis_index("c")
        sid = lax.axis_index("s")
        wid = cid * NS + sid
        base_row = wid * rows_per_worker
        r0 = sid * rpt
        for t in range(T):
            pltpu.sync_copy(zeros1.at[pl.ds(r0, rpt)],
                            accs[t].at[pl.ds(r0, rpt)])
        plsc.subcore_barrier()

        row_bytes = 4 * CH

        def _drain(sem, nbytes):
            pltpu.make_async_copy(zeros1.at[pl.ds(0, nbytes // 4)],
                                  drain_v.at[pl.ds(0, nbytes // 4)],
                                  sem).wait()

        D = 8

        def stage(st, carry):
            row0 = base_row + st * RS
            pltpu.sync_copy(src2d.at[pl.ds(row0, RS)], src_v)
            pltpu.sync_copy(dst2d.at[pl.ds(row0, RS)], dst_v)

            for k in range(D - 1):
                for t in range(T):
                    pltpu.async_copy(xts[t].at[src_v.at[k]], vals.at[k, t],
                                     gsem)

            def inner(j, c2):
                @pl.when(j > 0)
                def _():
                    _drain(ssem, T * row_bytes)

                @pl.when(j + D - 1 < RS)
                def _():
                    for t in range(T):
                        pltpu.async_copy(xts[t].at[src_v.at[j + D - 1]],
                                         vals.at[lax.rem(j + D - 1, D), t],
                                         gsem)

                _drain(gsem, T * row_bytes)
                for t in range(T):
                    pltpu.async_copy(vals.at[lax.rem(j, D), t],
                                     accs[t].at[dst_v.at[j]], ssem, add=True)
                return c2

            r = lax.fori_loop(0, RS, inner, carry)
            _drain(ssem, T * row_bytes)
            return r

        lax.fori_loop(0, n_stages, stage, 0)
        plsc.subcore_barrier()
        for t in range(T):
            pltpu.sync_copy(accs[t].at[pl.ds(r0, rpt)],
                            out.at[pl.ds((cid * T + t) * NP + r0, rpt)])

    return time_agg


def _make_scalar_agg_kernel(NP, n_rows):
    rows_per_worker = n_rows // NW
    n_stages = rows_per_worker // RS
    rpt = NP // NS

    @functools.partial(
        pl.kernel,
        out_type=jax.ShapeDtypeStruct((NC * NP,), F32),
        mesh=_mesh(),
        scratch_types=[
            pltpu.VMEM((RS, CH), jnp.int32),
            pltpu.VMEM((RS, CH), jnp.int32),
            pltpu.VMEM((8, CH), F32),
            pltpu.VMEM((CH,), F32),
            pltpu.VMEM_SHARED((NP,), F32),
            pltpu.SemaphoreType.DMA,
            pltpu.SemaphoreType.DMA,
        ],
    )
    def scal_agg(ys, src2d, dst2d, zeros1, out,
                 src_v, dst_v, vals, drain_v, b_sh, gsem, ssem):
        cid = lax.axis_index("c")
        sid = lax.axis_index("s")
        wid = cid * NS + sid
        base_row = wid * rows_per_worker
        r0 = sid * rpt
        pltpu.sync_copy(zeros1.at[pl.ds(r0, rpt)], b_sh.at[pl.ds(r0, rpt)])
        plsc.subcore_barrier()

        row_bytes = 4 * CH

        def _drain(sem, nbytes):
            pltpu.make_async_copy(zeros1.at[pl.ds(0, nbytes // 4)],
                                  drain_v.at[pl.ds(0, nbytes // 4)],
                                  sem).wait()

        D = 8

        def stage(st, carry):
            row0 = base_row + st * RS
            pltpu.sync_copy(src2d.at[pl.ds(row0, RS)], src_v)
            pltpu.sync_copy(dst2d.at[pl.ds(row0, RS)], dst_v)
            for k in range(D - 1):
                pltpu.async_copy(ys.at[src_v.at[k]], vals.at[k], gsem)

            def inner(j, c2):
                @pl.when(j > 0)
                def _():
                    _drain(ssem, row_bytes)

                @pl.when(j + D - 1 < RS)
                def _():
                    pltpu.async_copy(ys.at[src_v.at[j + D - 1]],
                                     vals.at[lax.rem(j + D - 1, D)], gsem)

                _drain(gsem, row_bytes)
                pltpu.async_copy(vals.at[lax.rem(j, D)],
                                 b_sh.at[dst_v.at[j]], ssem, add=True)
                return c2

            r = lax.fori_loop(0, RS, inner, carry)
            _drain(ssem, row_bytes)
            return r

        lax.fori_loop(0, n_stages, stage, 0)
        plsc.subcore_barrier()
        pltpu.sync_copy(b_sh.at[pl.ds(r0, rpt)],
                        out.at[pl.ds(cid * NP + r0, rpt)])

    return scal_agg


def _prep_body(d, xT, dinv_o, xs_o):
    deg = d[0:1, :] + d[1:2, :] + 1.0
    dinv = lax.rsqrt(deg)
    dinv_o[...] = dinv
    xs_o[...] = dinv * xT[...]


def _make_gru_body(T, H):
    def _gru_body(aT, xsT, dinv, w1c, b1c, wih, whh, bic, bhc, w2r, ys_o):
        dv = dinv[...]
        s = dv * (aT[0:T, :] + aT[T:2 * T, :] + xsT[...])
        h = jnp.zeros((H, s.shape[1]), F32)
        for t in range(T):
            g = jax.nn.relu(w1c[...] * s[t:t + 1, :] + b1c[...])
            ui = jnp.dot(wih[...], g, preferred_element_type=F32) + bic[...]
            uh = jnp.dot(whh[...], h, preferred_element_type=F32) + bhc[...]
            r = jax.nn.sigmoid(ui[0:H] + uh[0:H])
            z = jax.nn.sigmoid(ui[H:2 * H] + uh[H:2 * H])
            nt = jnp.tanh(ui[2 * H:3 * H] + r * uh[2 * H:3 * H])
            h = (1.0 - z) * nt + z * h
        y = jnp.dot(w2r[...], h, preferred_element_type=F32)
        ys_o[...] = dv * y
    return _gru_body


def _out_body(b, ysc, dinv, b2s, o):
    o[...] = dinv[...] * (b[0:1, :] + b[1:2, :] + ysc[...]) + b2s[...]


def _row_spec(r):
    return pl.BlockSpec((r, BLK), lambda i: (0, i))


def _full_spec(shape):
    return pl.BlockSpec(shape, lambda i: tuple(0 for _ in shape))


def kernel(x_seq, edge_index, W1, b1, Wih, Whh, bih, bhh, W2, b2):
    N, T = x_seq.shape
    E = edge_index.shape[1]
    H = Whh.shape[1]
    NP = ((N + 2047) // 2048) * 2048
    nblocks = NP // BLK
    n_rows = E // CH

    xT = jnp.pad(x_seq.T, ((0, 0), (0, NP - N)))
    src2d = edge_index[0].reshape(n_rows, CH)
    dst2d = edge_index[1].reshape(n_rows, CH)
    zeros1 = jnp.zeros((NP,), F32)
    w1c = W1.reshape(H, 1)
    b1c = b1.reshape(H, 1)
    bic = bih.reshape(3 * H, 1)
    bhc = bhh.reshape(3 * H, 1)
    w2r = W2.reshape(1, H)
    b2s = b2.reshape(1, 1)

    deg2 = _make_deg_kernel(NP, n_rows)(dst2d, zeros1).reshape(NC, NP)

    dinv, xsT = pl.pallas_call(
        _prep_body,
        grid=(nblocks,),
        in_specs=[_row_spec(NC), _row_spec(T)],
        out_specs=[_row_spec(1), _row_spec(T)],
        out_shape=[jax.ShapeDtypeStruct((1, NP), F32),
                   jax.ShapeDtypeStruct((T, NP), F32)],
    )(deg2, xT)

    xts = [xsT[t] for t in range(T)]
    aT = _make_time_agg_kernel(NP, T, n_rows)(
        *xts, src2d, dst2d, zeros1).reshape(NC * T, NP)

    ysc = pl.pallas_call(
        _make_gru_body(T, H),
        grid=(nblocks,),
        in_specs=[_row_spec(NC * T), _row_spec(T), _row_spec(1),
                  _full_spec((H, 1)), _full_spec((H, 1)),
                  _full_spec((3 * H, H)), _full_spec((3 * H, H)),
                  _full_spec((3 * H, 1)), _full_spec((3 * H, 1)),
                  _full_spec((1, H))],
        out_specs=_row_spec(1),
        out_shape=jax.ShapeDtypeStruct((1, NP), F32),
    )(aT, xsT, dinv, w1c, b1c, Wih, Whh, bic, bhc, w2r)

    b2p = _make_scalar_agg_kernel(NP, n_rows)(
        ysc.reshape(NP), src2d, dst2d, zeros1).reshape(NC, NP)

    o = pl.pallas_call(
        _out_body,
        grid=(nblocks,),
        in_specs=[_row_spec(NC), _row_spec(1), _row_spec(1),
                  _full_spec((1, 1))],
        out_specs=_row_spec(1),
        out_shape=jax.ShapeDtypeStruct((1, NP), F32),
    )(b2p, ysc, dinv, b2s)

    return o[:, :N]


# speedup vs baseline: 144.6783x; 1.0079x over previous
import functools

import jax
import jax.numpy as jnp
from jax import lax
from jax.experimental import pallas as pl
from jax.experimental.pallas import tpu as pltpu
from jax.experimental.pallas import tpu_sc as plsc

F32 = jnp.float32

NC = 2
NS = 16
NW = NC * NS
CH = 125
RS = 80
BLK = 12800


def _mesh():
    return plsc.VectorSubcoreMesh(
        core_axis_name="c", subcore_axis_name="s", num_cores=NC, num_subcores=NS
    )


def _make_deg_kernel(NP, n_rows):
    rows_per_worker = n_rows // NW
    n_stages = rows_per_worker // RS
    rpt = NP // NS

    @functools.partial(
        pl.kernel,
        out_type=jax.ShapeDtypeStruct((NC * NP,), F32),
        mesh=_mesh(),
        scratch_types=[
            pltpu.VMEM((RS, CH), jnp.int32),
            pltpu.VMEM((128,), F32),
            pltpu.VMEM((CH,), F32),
            pltpu.VMEM_SHARED((NP,), F32),
            pltpu.SemaphoreType.DMA,
        ],
    )
    def deg_kernel(dst2d, zeros1, out, idx_v, ones_v, drain_v, deg_sh, ssem):
        cid = lax.axis_index("c")
        sid = lax.axis_index("s")
        wid = cid * NS + sid
        base_row = wid * rows_per_worker
        r0 = sid * rpt
        pltpu.sync_copy(zeros1.at[pl.ds(r0, rpt)], deg_sh.at[pl.ds(r0, rpt)])
        for i in range(128 // 16):
            ones_v[pl.ds(i * 16, 16)] = jnp.ones((16,), F32)
        plsc.subcore_barrier()

        row_bytes = 4 * CH
        LAG = 8

        def _drain(nbytes):
            pltpu.make_async_copy(zeros1.at[pl.ds(0, nbytes // 4)],
                                  drain_v.at[pl.ds(0, nbytes // 4)],
                                  ssem).wait()

        def stage(st, carry):
            pltpu.sync_copy(dst2d.at[pl.ds(base_row + st * RS, RS)], idx_v)

            def inner(j, c2):
                @pl.when(j >= LAG)
                def _():
                    _drain(row_bytes)

                pltpu.async_copy(ones_v.at[pl.ds(0, CH)],
                                 deg_sh.at[idx_v.at[j]], ssem, add=True)
                return c2

            r = lax.fori_loop(0, RS, inner, carry)
            _drain(LAG * row_bytes)
            return r

        lax.fori_loop(0, n_stages, stage, 0)
        plsc.subcore_barrier()
        pltpu.sync_copy(deg_sh.at[pl.ds(r0, rpt)],
                        out.at[pl.ds(cid * NP + r0, rpt)])

    return deg_kernel


def _make_time_agg_kernel(NP, T, n_rows):
    rows_per_worker = n_rows // NW
    n_stages = rows_per_worker // RS
    rpt = NP // NS

    @functools.partial(
        pl.kernel,
        out_type=jax.ShapeDtypeStruct((NC * T * NP,), F32),
        mesh=_mesh(),
        scratch_types=[
            pltpu.VMEM((RS, CH), jnp.int32),
            pltpu.VMEM((RS, CH), jnp.int32),
            pltpu.VMEM((12, T, CH), F32),
            pltpu.VMEM((T * CH,), F32),
            [pltpu.VMEM_SHARED((NP,), F32) for _ in range(T)],
            pltpu.SemaphoreType.DMA,
            pltpu.SemaphoreType.DMA,
        ],
    )
    def time_agg(*refs):
        xts = refs[0:T]
        src2d, dst2d, zeros1, out = refs[T:T + 4]
        src_v, dst_v, vals, drain_v, accs, gsem, ssem = refs[T + 4:]
        cid = lax.axis_index("c")
        sid = lax.axis_index("s")
        wid = cid * NS + sid
        base_row = wid * rows_per_worker
        r0 = sid * rpt
        for t in range(T):
            pltpu.sync_copy(zeros1.at[pl.ds(r0, rpt)],
                            accs[t].at[pl.ds(r0, rpt)])
        plsc.subcore_barrier()

        row_bytes = 4 * CH

        def _drain(sem, nbytes):
            pltpu.make_async_copy(zeros1.at[pl.ds(0, nbytes // 4)],
                                  drain_v.at[pl.ds(0, nbytes // 4)],
                                  sem).wait()

        D = 12

        def stage(st, carry):
            row0 = base_row + st * RS
            pltpu.sync_copy(src2d.at[pl.ds(row0, RS)], src_v)
            pltpu.sync_copy(dst2d.at[pl.ds(row0, RS)], dst_v)

            for k in range(D - 1):
                for t in range(T):
                    pltpu.async_copy(xts[t].at[src_v.at[k]], vals.at[k, t],
                                     gsem)

            def inner(j, c2):
                @pl.when(j > 0)
                def _():
                    _drain(ssem, T * row_bytes)

                @pl.when(j + D - 1 < RS)
                def _():
                    for t in range(T):
                        pltpu.async_copy(xts[t].at[src_v.at[j + D - 1]],
                                         vals.at[lax.rem(j + D - 1, D), t],
                                         gsem)

                _drain(gsem, T * row_bytes)
                for t in range(T):
                    pltpu.async_copy(vals.at[lax.rem(j, D), t],
                                     accs[t].at[dst_v.at[j]], ssem, add=True)
                return c2

            r = lax.fori_loop(0, RS, inner, carry)
            _drain(ssem, T * row_bytes)
            return r

        lax.fori_loop(0, n_stages, stage, 0)
        plsc.subcore_barrier()
        for t in range(T):
            pltpu.sync_copy(accs[t].at[pl.ds(r0, rpt)],
                            out.at[pl.ds((cid * T + t) * NP + r0, rpt)])

    return time_agg


def _make_scalar_agg_kernel(NP, n_rows):
    rows_per_worker = n_rows // NW
    n_stages = rows_per_worker // RS
    rpt = NP // NS

    @functools.partial(
        pl.kernel,
        out_type=jax.ShapeDtypeStruct((NC * NP,), F32),
        mesh=_mesh(),
        scratch_types=[
            pltpu.VMEM((RS, CH), jnp.int32),
            pltpu.VMEM((RS, CH), jnp.int32),
            pltpu.VMEM((8, CH), F32),
            pltpu.VMEM((CH,), F32),
            pltpu.VMEM_SHARED((NP,), F32),
            pltpu.SemaphoreType.DMA,
            pltpu.SemaphoreType.DMA,
        ],
    )
    def scal_agg(ys, src2d, dst2d, zeros1, out,
                 src_v, dst_v, vals, drain_v, b_sh, gsem, ssem):
        cid = lax.axis_index("c")
        sid = lax.axis_index("s")
        wid = cid * NS + sid
        base_row = wid * rows_per_worker
        r0 = sid * rpt
        pltpu.sync_copy(zeros1.at[pl.ds(r0, rpt)], b_sh.at[pl.ds(r0, rpt)])
        plsc.subcore_barrier()

        row_bytes = 4 * CH

        def _drain(sem, nbytes):
            pltpu.make_async_copy(zeros1.at[pl.ds(0, nbytes // 4)],
                                  drain_v.at[pl.ds(0, nbytes // 4)],
                                  sem).wait()

        D = 8

        def stage(st, carry):
            row0 = base_row + st * RS
            pltpu.sync_copy(src2d.at[pl.ds(row0, RS)], src_v)
            pltpu.sync_copy(dst2d.at[pl.ds(row0, RS)], dst_v)
            for k in range(D - 1):
                pltpu.async_copy(ys.at[src_v.at[k]], vals.at[k], gsem)

            def inner(j, c2):
                @pl.when(j > 0)
                def _():
                    _drain(ssem, row_bytes)

                @pl.when(j + D - 1 < RS)
                def _():
                    pltpu.async_copy(ys.at[src_v.at[j + D - 1]],
                                     vals.at[lax.rem(j + D - 1, D)], gsem)

                _drain(gsem, row_bytes)
                pltpu.async_copy(vals.at[lax.rem(j, D)],
                                 b_sh.at[dst_v.at[j]], ssem, add=True)
                return c2

            r = lax.fori_loop(0, RS, inner, carry)
            _drain(ssem, row_bytes)
            return r

        lax.fori_loop(0, n_stages, stage, 0)
        plsc.subcore_barrier()
        pltpu.sync_copy(b_sh.at[pl.ds(r0, rpt)],
                        out.at[pl.ds(cid * NP + r0, rpt)])

    return scal_agg


def _prep_body(d, xT, dinv_o, xs_o):
    deg = d[0:1, :] + d[1:2, :] + 1.0
    dinv = lax.rsqrt(deg)
    dinv_o[...] = dinv
    xs_o[...] = dinv * xT[...]


def _make_gru_body(T, H):
    def _gru_body(aT, xsT, dinv, w1c, b1c, wih, whh, bic, bhc, w2r, ys_o):
        dv = dinv[...]
        s = dv * (aT[0:T, :] + aT[T:2 * T, :] + xsT[...])
        h = jnp.zeros((H, s.shape[1]), F32)
        for t in range(T):
            g = jax.nn.relu(w1c[...] * s[t:t + 1, :] + b1c[...])
            ui = jnp.dot(wih[...], g, preferred_element_type=F32) + bic[...]
            uh = jnp.dot(whh[...], h, preferred_element_type=F32) + bhc[...]
            r = jax.nn.sigmoid(ui[0:H] + uh[0:H])
            z = jax.nn.sigmoid(ui[H:2 * H] + uh[H:2 * H])
            nt = jnp.tanh(ui[2 * H:3 * H] + r * uh[2 * H:3 * H])
            h = (1.0 - z) * nt + z * h
        y = jnp.dot(w2r[...], h, preferred_element_type=F32)
        ys_o[...] = dv * y
    return _gru_body


def _out_body(b, ysc, dinv, b2s, o):
    o[...] = dinv[...] * (b[0:1, :] + b[1:2, :] + ysc[...]) + b2s[...]


def _row_spec(r):
    return pl.BlockSpec((r, BLK), lambda i: (0, i))


def _full_spec(shape):
    return pl.BlockSpec(shape, lambda i: tuple(0 for _ in shape))


def kernel(x_seq, edge_index, W1, b1, Wih, Whh, bih, bhh, W2, b2):
    N, T = x_seq.shape
    E = edge_index.shape[1]
    H = Whh.shape[1]
    NP = ((N + 2047) // 2048) * 2048
    nblocks = NP // BLK
    n_rows = E // CH

    xT = jnp.pad(x_seq.T, ((0, 0), (0, NP - N)))
    src2d = edge_index[0].reshape(n_rows, CH)
    dst2d = edge_index[1].reshape(n_rows, CH)
    zeros1 = jnp.zeros((NP,), F32)
    w1c = W1.reshape(H, 1)
    b1c = b1.reshape(H, 1)
    bic = bih.reshape(3 * H, 1)
    bhc = bhh.reshape(3 * H, 1)
    w2r = W2.reshape(1, H)
    b2s = b2.reshape(1, 1)

    deg2 = _make_deg_kernel(NP, n_rows)(dst2d, zeros1).reshape(NC, NP)

    dinv, xsT = pl.pallas_call(
        _prep_body,
        grid=(nblocks,),
        in_specs=[_row_spec(NC), _row_spec(T)],
        out_specs=[_row_spec(1), _row_spec(T)],
        out_shape=[jax.ShapeDtypeStruct((1, NP), F32),
                   jax.ShapeDtypeStruct((T, NP), F32)],
    )(deg2, xT)

    xts = [xsT[t] for t in range(T)]
    aT = _make_time_agg_kernel(NP, T, n_rows)(
        *xts, src2d, dst2d, zeros1).reshape(NC * T, NP)

    ysc = pl.pallas_call(
        _make_gru_body(T, H),
        grid=(nblocks,),
        in_specs=[_row_spec(NC * T), _row_spec(T), _row_spec(1),
                  _full_spec((H, 1)), _full_spec((H, 1)),
                  _full_spec((3 * H, H)), _full_spec((3 * H, H)),
                  _full_spec((3 * H, 1)), _full_spec((3 * H, 1)),
                  _full_spec((1, H))],
        out_specs=_row_spec(1),
        out_shape=jax.ShapeDtypeStruct((1, NP), F32),
    )(aT, xsT, dinv, w1c, b1c, Wih, Whh, bic, bhc, w2r)

    b2p = _make_scalar_agg_kernel(NP, n_rows)(
        ysc.reshape(NP), src2d, dst2d, zeros1).reshape(NC, NP)

    o = pl.pallas_call(
        _out_body,
        grid=(nblocks,),
        in_specs=[_row_spec(NC), _row_spec(1), _row_spec(1),
                  _full_spec((1, 1))],
        out_specs=_row_spec(1),
        out_shape=jax.ShapeDtypeStruct((1, NP), F32),
    )(b2p, ysc, dinv, b2s)

    return o[:, :N]


# speedup vs baseline: 188.4053x; 1.3022x over previous
import functools

import jax
import jax.numpy as jnp
from jax import lax
from jax.experimental import pallas as pl
from jax.experimental.pallas import tpu as pltpu
from jax.experimental.pallas import tpu_sc as plsc

F32 = jnp.float32

NC = 2
NS = 16
NW = NC * NS
CH = 125
RS = 80
BLK = 12800


def _mesh():
    return plsc.VectorSubcoreMesh(
        core_axis_name="c", subcore_axis_name="s", num_cores=NC, num_subcores=NS
    )


def _make_deg_kernel(NP, n_rows):
    rows_per_worker = n_rows // NW
    n_stages = rows_per_worker // RS
    rpt = NP // NS

    @functools.partial(
        pl.kernel,
        out_type=jax.ShapeDtypeStruct((NC * NP,), F32),
        mesh=_mesh(),
        scratch_types=[
            pltpu.VMEM((RS, CH), jnp.int32),
            pltpu.VMEM((128,), F32),
            pltpu.VMEM((CH,), F32),
            pltpu.VMEM_SHARED((NP,), F32),
            pltpu.SemaphoreType.DMA,
        ],
    )
    def deg_kernel(dst2d, zeros1, out, idx_v, ones_v, drain_v, deg_sh, ssem):
        cid = lax.axis_index("c")
        sid = lax.axis_index("s")
        wid = cid * NS + sid
        base_row = wid * rows_per_worker
        r0 = sid * rpt
        pltpu.sync_copy(zeros1.at[pl.ds(r0, rpt)], deg_sh.at[pl.ds(r0, rpt)])
        for i in range(128 // 16):
            ones_v[pl.ds(i * 16, 16)] = jnp.ones((16,), F32)
        plsc.subcore_barrier()

        row_bytes = 4 * CH
        LAG = 8

        def _drain(nbytes):
            pltpu.make_async_copy(zeros1.at[pl.ds(0, nbytes // 4)],
                                  drain_v.at[pl.ds(0, nbytes // 4)],
                                  ssem).wait()

        def stage(st, carry):
            pltpu.sync_copy(dst2d.at[pl.ds(base_row + st * RS, RS)], idx_v)

            def inner(j, c2):
                @pl.when(j >= LAG)
                def _():
                    _drain(row_bytes)

                pltpu.async_copy(ones_v.at[pl.ds(0, CH)],
                                 deg_sh.at[idx_v.at[j]], ssem, add=True)
                return c2

            r = lax.fori_loop(0, RS, inner, carry)
            _drain(LAG * row_bytes)
            return r

        lax.fori_loop(0, n_stages, stage, 0)
        plsc.subcore_barrier()
        pltpu.sync_copy(deg_sh.at[pl.ds(r0, rpt)],
                        out.at[pl.ds(cid * NP + r0, rpt)])

    return deg_kernel


def _make_time_agg_kernel(NP, T, n_rows):
    rows_per_worker = n_rows // NW
    n_stages = rows_per_worker // RS
    rpt = NP // NS

    @functools.partial(
        pl.kernel,
        out_type=jax.ShapeDtypeStruct((NC * T * NP,), F32),
        mesh=_mesh(),
        scratch_types=[
            pltpu.VMEM((RS, CH), jnp.int32),
            pltpu.VMEM((RS, CH), jnp.int32),
            pltpu.VMEM((12, T, CH), F32),
            pltpu.VMEM((T * CH,), F32),
            [pltpu.VMEM_SHARED((NP,), F32) for _ in range(T)],
            [pltpu.VMEM_SHARED((NP,), F32) for _ in range(T // 2)],
            pltpu.SemaphoreType.DMA,
            pltpu.SemaphoreType.DMA,
            pltpu.SemaphoreType.DMA,
        ],
    )
    def time_agg(*refs):
        xts = refs[0:T]
        src2d, dst2d, zeros1, out = refs[T:T + 4]
        (src_v, dst_v, vals, drain_v, accs, xsh,
         gsem, gsem2, ssem) = refs[T + 4:]
        cid = lax.axis_index("c")
        sid = lax.axis_index("s")
        wid = cid * NS + sid
        base_row = wid * rows_per_worker
        r0 = sid * rpt
        for t in range(T):
            pltpu.sync_copy(zeros1.at[pl.ds(r0, rpt)],
                            accs[t].at[pl.ds(r0, rpt)])
        for t in range(T // 2, T):
            pltpu.sync_copy(xts[t].at[pl.ds(r0, rpt)],
                            xsh[t - T // 2].at[pl.ds(r0, rpt)])
        plsc.subcore_barrier()

        def src_arr(t):
            if t < T // 2:
                return xts[t], gsem
            return xsh[t - T // 2], gsem2

        row_bytes = 4 * CH

        def _drain(sem, nbytes):
            pltpu.make_async_copy(zeros1.at[pl.ds(0, nbytes // 4)],
                                  drain_v.at[pl.ds(0, nbytes // 4)],
                                  sem).wait()

        D = 12

        def stage(st, carry):
            row0 = base_row + st * RS
            pltpu.sync_copy(src2d.at[pl.ds(row0, RS)], src_v)
            pltpu.sync_copy(dst2d.at[pl.ds(row0, RS)], dst_v)

            for k in range(D - 1):
                for t in range(T):
                    arr, sem = src_arr(t)
                    pltpu.async_copy(arr.at[src_v.at[k]], vals.at[k, t], sem)

            def inner(j, c2):
                @pl.when(j > 0)
                def _():
                    _drain(ssem, T * row_bytes)

                @pl.when(j + D - 1 < RS)
                def _():
                    for t in range(T):
                        arr, sem = src_arr(t)
                        pltpu.async_copy(arr.at[src_v.at[j + D - 1]],
                                         vals.at[lax.rem(j + D - 1, D), t],
                                         sem)

                _drain(gsem, (T // 2) * row_bytes)
                _drain(gsem2, (T // 2) * row_bytes)
                for t in range(T):
                    pltpu.async_copy(vals.at[lax.rem(j, D), t],
                                     accs[t].at[dst_v.at[j]], ssem, add=True)
                return c2

            r = lax.fori_loop(0, RS, inner, carry)
            _drain(ssem, T * row_bytes)
            return r

        lax.fori_loop(0, n_stages, stage, 0)
        plsc.subcore_barrier()
        for t in range(T):
            pltpu.sync_copy(accs[t].at[pl.ds(r0, rpt)],
                            out.at[pl.ds((cid * T + t) * NP + r0, rpt)])

    return time_agg


def _make_scalar_agg_kernel(NP, n_rows):
    rows_per_worker = n_rows // NW
    n_stages = rows_per_worker // RS
    rpt = NP // NS

    @functools.partial(
        pl.kernel,
        out_type=jax.ShapeDtypeStruct((NC * NP,), F32),
        mesh=_mesh(),
        scratch_types=[
            pltpu.VMEM((RS, CH), jnp.int32),
            pltpu.VMEM((RS, CH), jnp.int32),
            pltpu.VMEM((8, CH), F32),
            pltpu.VMEM((CH,), F32),
            pltpu.VMEM_SHARED((NP,), F32),
            pltpu.SemaphoreType.DMA,
            pltpu.SemaphoreType.DMA,
        ],
    )
    def scal_agg(ys, src2d, dst2d, zeros1, out,
                 src_v, dst_v, vals, drain_v, b_sh, gsem, ssem):
        cid = lax.axis_index("c")
        sid = lax.axis_index("s")
        wid = cid * NS + sid
        base_row = wid * rows_per_worker
        r0 = sid * rpt
        pltpu.sync_copy(zeros1.at[pl.ds(r0, rpt)], b_sh.at[pl.ds(r0, rpt)])
        plsc.subcore_barrier()

        row_bytes = 4 * CH

        def _drain(sem, nbytes):
            pltpu.make_async_copy(zeros1.at[pl.ds(0, nbytes // 4)],
                                  drain_v.at[pl.ds(0, nbytes // 4)],
                                  sem).wait()

        D = 8

        def stage(st, carry):
            row0 = base_row + st * RS
            pltpu.sync_copy(src2d.at[pl.ds(row0, RS)], src_v)
            pltpu.sync_copy(dst2d.at[pl.ds(row0, RS)], dst_v)
            for k in range(D - 1):
                pltpu.async_copy(ys.at[src_v.at[k]], vals.at[k], gsem)

            def inner(j, c2):
                @pl.when(j > 0)
                def _():
                    _drain(ssem, row_bytes)

                @pl.when(j + D - 1 < RS)
                def _():
                    pltpu.async_copy(ys.at[src_v.at[j + D - 1]],
                                     vals.at[lax.rem(j + D - 1, D)], gsem)

                _drain(gsem, row_bytes)
                pltpu.async_copy(vals.at[lax.rem(j, D)],
                                 b_sh.at[dst_v.at[j]], ssem, add=True)
                return c2

            r = lax.fori_loop(0, RS, inner, carry)
            _drain(ssem, row_bytes)
            return r

        lax.fori_loop(0, n_stages, stage, 0)
        plsc.subcore_barrier()
        pltpu.sync_copy(b_sh.at[pl.ds(r0, rpt)],
                        out.at[pl.ds(cid * NP + r0, rpt)])

    return scal_agg


def _prep_body(d, xT, dinv_o, xs_o):
    deg = d[0:1, :] + d[1:2, :] + 1.0
    dinv = lax.rsqrt(deg)
    dinv_o[...] = dinv
    xs_o[...] = dinv * xT[...]


def _make_gru_body(T, H):
    def _gru_body(aT, xsT, dinv, w1c, b1c, wih, whh, bic, bhc, w2r, ys_o):
        dv = dinv[...]
        s = dv * (aT[0:T, :] + aT[T:2 * T, :] + xsT[...])
        h = jnp.zeros((H, s.shape[1]), F32)
        for t in range(T):
            g = jax.nn.relu(w1c[...] * s[t:t + 1, :] + b1c[...])
            ui = jnp.dot(wih[...], g, preferred_element_type=F32) + bic[...]
            uh = jnp.dot(whh[...], h, preferred_element_type=F32) + bhc[...]
            r = jax.nn.sigmoid(ui[0:H] + uh[0:H])
            z = jax.nn.sigmoid(ui[H:2 * H] + uh[H:2 * H])
            nt = jnp.tanh(ui[2 * H:3 * H] + r * uh[2 * H:3 * H])
            h = (1.0 - z) * nt + z * h
        y = jnp.dot(w2r[...], h, preferred_element_type=F32)
        ys_o[...] = dv * y
    return _gru_body


def _out_body(b, ysc, dinv, b2s, o):
    o[...] = dinv[...] * (b[0:1, :] + b[1:2, :] + ysc[...]) + b2s[...]


def _row_spec(r):
    return pl.BlockSpec((r, BLK), lambda i: (0, i))


def _full_spec(shape):
    return pl.BlockSpec(shape, lambda i: tuple(0 for _ in shape))


def kernel(x_seq, edge_index, W1, b1, Wih, Whh, bih, bhh, W2, b2):
    N, T = x_seq.shape
    E = edge_index.shape[1]
    H = Whh.shape[1]
    NP = ((N + 2047) // 2048) * 2048
    nblocks = NP // BLK
    n_rows = E // CH

    xT = jnp.pad(x_seq.T, ((0, 0), (0, NP - N)))
    src2d = edge_index[0].reshape(n_rows, CH)
    dst2d = edge_index[1].reshape(n_rows, CH)
    zeros1 = jnp.zeros((NP,), F32)
    w1c = W1.reshape(H, 1)
    b1c = b1.reshape(H, 1)
    bic = bih.reshape(3 * H, 1)
    bhc = bhh.reshape(3 * H, 1)
    w2r = W2.reshape(1, H)
    b2s = b2.reshape(1, 1)

    deg2 = _make_deg_kernel(NP, n_rows)(dst2d, zeros1).reshape(NC, NP)

    dinv, xsT = pl.pallas_call(
        _prep_body,
        grid=(nblocks,),
        in_specs=[_row_spec(NC), _row_spec(T)],
        out_specs=[_row_spec(1), _row_spec(T)],
        out_shape=[jax.ShapeDtypeStruct((1, NP), F32),
                   jax.ShapeDtypeStruct((T, NP), F32)],
    )(deg2, xT)

    xts = [xsT[t] for t in range(T)]
    aT = _make_time_agg_kernel(NP, T, n_rows)(
        *xts, src2d, dst2d, zeros1).reshape(NC * T, NP)

    ysc = pl.pallas_call(
        _make_gru_body(T, H),
        grid=(nblocks,),
        in_specs=[_row_spec(NC * T), _row_spec(T), _row_spec(1),
                  _full_spec((H, 1)), _full_spec((H, 1)),
                  _full_spec((3 * H, H)), _full_spec((3 * H, H)),
                  _full_spec((3 * H, 1)), _full_spec((3 * H, 1)),
                  _full_spec((1, H))],
        out_specs=_row_spec(1),
        out_shape=jax.ShapeDtypeStruct((1, NP), F32),
    )(aT, xsT, dinv, w1c, b1c, Wih, Whh, bic, bhc, w2r)

    b2p = _make_scalar_agg_kernel(NP, n_rows)(
        ysc.reshape(NP), src2d, dst2d, zeros1).reshape(NC, NP)

    o = pl.pallas_call(
        _out_body,
        grid=(nblocks,),
        in_specs=[_row_spec(NC), _row_spec(1), _row_spec(1),
                  _full_spec((1, 1))],
        out_specs=_row_spec(1),
        out_shape=jax.ShapeDtypeStruct((1, NP), F32),
    )(b2p, ysc, dinv, b2s)

    return o[:, :N]


# speedup vs baseline: 196.7929x; 1.0445x over previous
import functools

import jax
import jax.numpy as jnp
from jax import lax
from jax.experimental import pallas as pl
from jax.experimental.pallas import tpu as pltpu
from jax.experimental.pallas import tpu_sc as plsc

F32 = jnp.float32

NC = 2
NS = 16
NW = NC * NS
CH = 125
RS = 80
BLK = 12800


def _mesh():
    return plsc.VectorSubcoreMesh(
        core_axis_name="c", subcore_axis_name="s", num_cores=NC, num_subcores=NS
    )


def _make_deg_kernel(NP, n_rows):
    rows_per_worker = n_rows // NW
    n_stages = rows_per_worker // RS
    rpt = NP // NS

    @functools.partial(
        pl.kernel,
        out_type=jax.ShapeDtypeStruct((NC * NP,), F32),
        mesh=_mesh(),
        scratch_types=[
            pltpu.VMEM((RS, CH), jnp.int32),
            pltpu.VMEM((128,), F32),
            pltpu.VMEM((CH,), F32),
            pltpu.VMEM_SHARED((NP,), F32),
            pltpu.SemaphoreType.DMA,
        ],
    )
    def deg_kernel(dst2d, zeros1, out, idx_v, ones_v, drain_v, deg_sh, ssem):
        cid = lax.axis_index("c")
        sid = lax.axis_index("s")
        wid = cid * NS + sid
        base_row = wid * rows_per_worker
        r0 = sid * rpt
        pltpu.sync_copy(zeros1.at[pl.ds(r0, rpt)], deg_sh.at[pl.ds(r0, rpt)])
        for i in range(128 // 16):
            ones_v[pl.ds(i * 16, 16)] = jnp.ones((16,), F32)
        plsc.subcore_barrier()

        row_bytes = 4 * CH
        LAG = 8

        def _drain(nbytes):
            pltpu.make_async_copy(zeros1.at[pl.ds(0, nbytes // 4)],
                                  drain_v.at[pl.ds(0, nbytes // 4)],
                                  ssem).wait()

        def stage(st, carry):
            pltpu.sync_copy(dst2d.at[pl.ds(base_row + st * RS, RS)], idx_v)

            def inner(j, c2):
                @pl.when(j >= LAG)
                def _():
                    _drain(row_bytes)

                pltpu.async_copy(ones_v.at[pl.ds(0, CH)],
                                 deg_sh.at[idx_v.at[j]], ssem, add=True)
                return c2

            r = lax.fori_loop(0, RS, inner, carry)
            _drain(LAG * row_bytes)
            return r

        lax.fori_loop(0, n_stages, stage, 0)
        plsc.subcore_barrier()
        pltpu.sync_copy(deg_sh.at[pl.ds(r0, rpt)],
                        out.at[pl.ds(cid * NP + r0, rpt)])

    return deg_kernel


def _make_time_agg_kernel(NP, T, n_rows):
    rows_per_worker = n_rows // NW
    n_stages = rows_per_worker // RS
    rpt = NP // NS

    @functools.partial(
        pl.kernel,
        out_type=jax.ShapeDtypeStruct((NC * T * NP,), F32),
        mesh=_mesh(),
        scratch_types=[
            pltpu.VMEM((RS, CH), jnp.int32),
            pltpu.VMEM((RS, CH), jnp.int32),
            pltpu.VMEM((12, T, CH), F32),
            pltpu.VMEM((T * CH,), F32),
            [pltpu.VMEM_SHARED((NP,), F32) for _ in range(T)],
            [pltpu.VMEM_SHARED((NP,), F32) for _ in range(T - T // 3)],
            pltpu.SemaphoreType.DMA,
            pltpu.SemaphoreType.DMA,
            pltpu.SemaphoreType.DMA,
        ],
    )
    def time_agg(*refs):
        xts = refs[0:T]
        src2d, dst2d, zeros1, out = refs[T:T + 4]
        (src_v, dst_v, vals, drain_v, accs, xsh,
         gsem, gsem2, ssem) = refs[T + 4:]
        cid = lax.axis_index("c")
        sid = lax.axis_index("s")
        wid = cid * NS + sid
        base_row = wid * rows_per_worker
        r0 = sid * rpt
        for t in range(T):
            pltpu.sync_copy(zeros1.at[pl.ds(r0, rpt)],
                            accs[t].at[pl.ds(r0, rpt)])
        for t in range(T // 3, T):
            pltpu.sync_copy(xts[t].at[pl.ds(r0, rpt)],
                            xsh[t - T // 3].at[pl.ds(r0, rpt)])
        plsc.subcore_barrier()

        def src_arr(t):
            if t < T // 3:
                return xts[t], gsem
            return xsh[t - T // 3], gsem2

        row_bytes = 4 * CH

        def _drain(sem, nbytes):
            pltpu.make_async_copy(zeros1.at[pl.ds(0, nbytes // 4)],
                                  drain_v.at[pl.ds(0, nbytes // 4)],
                                  sem).wait()

        D = 12

        def stage(st, carry):
            row0 = base_row + st * RS
            pltpu.sync_copy(src2d.at[pl.ds(row0, RS)], src_v)
            pltpu.sync_copy(dst2d.at[pl.ds(row0, RS)], dst_v)

            for k in range(D - 1):
                for t in range(T):
                    arr, sem = src_arr(t)
                    pltpu.async_copy(arr.at[src_v.at[k]], vals.at[k, t], sem)

            def inner(j, c2):
                @pl.when(j > 0)
                def _():
                    _drain(ssem, T * row_bytes)

                @pl.when(j + D - 1 < RS)
                def _():
                    for t in range(T):
                        arr, sem = src_arr(t)
                        pltpu.async_copy(arr.at[src_v.at[j + D - 1]],
                                         vals.at[lax.rem(j + D - 1, D), t],
                                         sem)

                _drain(gsem, (T // 3) * row_bytes)
                _drain(gsem2, (T - T // 3) * row_bytes)
                for t in range(T):
                    pltpu.async_copy(vals.at[lax.rem(j, D), t],
                                     accs[t].at[dst_v.at[j]], ssem, add=True)
                return c2

            r = lax.fori_loop(0, RS, inner, carry)
            _drain(ssem, T * row_bytes)
            return r

        lax.fori_loop(0, n_stages, stage, 0)
        plsc.subcore_barrier()
        for t in range(T):
            pltpu.sync_copy(accs[t].at[pl.ds(r0, rpt)],
                            out.at[pl.ds((cid * T + t) * NP + r0, rpt)])

    return time_agg


def _make_scalar_agg_kernel(NP, n_rows):
    rows_per_worker = n_rows // NW
    n_stages = rows_per_worker // RS
    rpt = NP // NS

    @functools.partial(
        pl.kernel,
        out_type=jax.ShapeDtypeStruct((NC * NP,), F32),
        mesh=_mesh(),
        scratch_types=[
            pltpu.VMEM((RS, CH), jnp.int32),
            pltpu.VMEM((RS, CH), jnp.int32),
            pltpu.VMEM((8, CH), F32),
            pltpu.VMEM((CH,), F32),
            pltpu.VMEM_SHARED((NP,), F32),
            pltpu.VMEM_SHARED((NP,), F32),
            pltpu.SemaphoreType.DMA,
            pltpu.SemaphoreType.DMA,
        ],
    )
    def scal_agg(ys, src2d, dst2d, zeros1, out,
                 src_v, dst_v, vals, drain_v, b_sh, ysh, gsem, ssem):
        cid = lax.axis_index("c")
        sid = lax.axis_index("s")
        wid = cid * NS + sid
        base_row = wid * rows_per_worker
        r0 = sid * rpt
        pltpu.sync_copy(zeros1.at[pl.ds(r0, rpt)], b_sh.at[pl.ds(r0, rpt)])
        pltpu.sync_copy(ys.at[pl.ds(r0, rpt)], ysh.at[pl.ds(r0, rpt)])
        plsc.subcore_barrier()

        row_bytes = 4 * CH

        def _drain(sem, nbytes):
            pltpu.make_async_copy(zeros1.at[pl.ds(0, nbytes // 4)],
                                  drain_v.at[pl.ds(0, nbytes // 4)],
                                  sem).wait()

        D = 8

        def stage(st, carry):
            row0 = base_row + st * RS
            pltpu.sync_copy(src2d.at[pl.ds(row0, RS)], src_v)
            pltpu.sync_copy(dst2d.at[pl.ds(row0, RS)], dst_v)
            for k in range(D - 1):
                pltpu.async_copy(ysh.at[src_v.at[k]], vals.at[k], gsem)

            def inner(j, c2):
                @pl.when(j > 0)
                def _():
                    _drain(ssem, row_bytes)

                @pl.when(j + D - 1 < RS)
                def _():
                    pltpu.async_copy(ysh.at[src_v.at[j + D - 1]],
                                     vals.at[lax.rem(j + D - 1, D)], gsem)

                _drain(gsem, row_bytes)
                pltpu.async_copy(vals.at[lax.rem(j, D)],
                                 b_sh.at[dst_v.at[j]], ssem, add=True)
                return c2

            r = lax.fori_loop(0, RS, inner, carry)
            _drain(ssem, row_bytes)
            return r

        lax.fori_loop(0, n_stages, stage, 0)
        plsc.subcore_barrier()
        pltpu.sync_copy(b_sh.at[pl.ds(r0, rpt)],
                        out.at[pl.ds(cid * NP + r0, rpt)])

    return scal_agg


def _prep_body(d, xT, dinv_o, xs_o):
    deg = d[0:1, :] + d[1:2, :] + 1.0
    dinv = lax.rsqrt(deg)
    dinv_o[...] = dinv
    xs_o[...] = dinv * xT[...]


def _make_gru_body(T, H):
    def _gru_body(aT, xsT, dinv, w1c, b1c, wih, whh, bic, bhc, w2r, ys_o):
        dv = dinv[...]
        s = dv * (aT[0:T, :] + aT[T:2 * T, :] + xsT[...])
        h = jnp.zeros((H, s.shape[1]), F32)
        for t in range(T):
            g = jax.nn.relu(w1c[...] * s[t:t + 1, :] + b1c[...])
            ui = jnp.dot(wih[...], g, preferred_element_type=F32) + bic[...]
            uh = jnp.dot(whh[...], h, preferred_element_type=F32) + bhc[...]
            r = jax.nn.sigmoid(ui[0:H] + uh[0:H])
            z = jax.nn.sigmoid(ui[H:2 * H] + uh[H:2 * H])
            nt = jnp.tanh(ui[2 * H:3 * H] + r * uh[2 * H:3 * H])
            h = (1.0 - z) * nt + z * h
        y = jnp.dot(w2r[...], h, preferred_element_type=F32)
        ys_o[...] = dv * y
    return _gru_body


def _out_body(b, ysc, dinv, b2s, o):
    o[...] = dinv[...] * (b[0:1, :] + b[1:2, :] + ysc[...]) + b2s[...]


def _row_spec(r):
    return pl.BlockSpec((r, BLK), lambda i: (0, i))


def _full_spec(shape):
    return pl.BlockSpec(shape, lambda i: tuple(0 for _ in shape))


def kernel(x_seq, edge_index, W1, b1, Wih, Whh, bih, bhh, W2, b2):
    N, T = x_seq.shape
    E = edge_index.shape[1]
    H = Whh.shape[1]
    NP = ((N + 2047) // 2048) * 2048
    nblocks = NP // BLK
    n_rows = E // CH

    xT = jnp.pad(x_seq.T, ((0, 0), (0, NP - N)))
    src2d = edge_index[0].reshape(n_rows, CH)
    dst2d = edge_index[1].reshape(n_rows, CH)
    zeros1 = jnp.zeros((NP,), F32)
    w1c = W1.reshape(H, 1)
    b1c = b1.reshape(H, 1)
    bic = bih.reshape(3 * H, 1)
    bhc = bhh.reshape(3 * H, 1)
    w2r = W2.reshape(1, H)
    b2s = b2.reshape(1, 1)

    deg2 = _make_deg_kernel(NP, n_rows)(dst2d, zeros1).reshape(NC, NP)

    dinv, xsT = pl.pallas_call(
        _prep_body,
        grid=(nblocks,),
        in_specs=[_row_spec(NC), _row_spec(T)],
        out_specs=[_row_spec(1), _row_spec(T)],
        out_shape=[jax.ShapeDtypeStruct((1, NP), F32),
                   jax.ShapeDtypeStruct((T, NP), F32)],
    )(deg2, xT)

    xts = [xsT[t] for t in range(T)]
    aT = _make_time_agg_kernel(NP, T, n_rows)(
        *xts, src2d, dst2d, zeros1).reshape(NC * T, NP)

    ysc = pl.pallas_call(
        _make_gru_body(T, H),
        grid=(nblocks,),
        in_specs=[_row_spec(NC * T), _row_spec(T), _row_spec(1),
                  _full_spec((H, 1)), _full_spec((H, 1)),
                  _full_spec((3 * H, H)), _full_spec((3 * H, H)),
                  _full_spec((3 * H, 1)), _full_spec((3 * H, 1)),
                  _full_spec((1, H))],
        out_specs=_row_spec(1),
        out_shape=jax.ShapeDtypeStruct((1, NP), F32),
    )(aT, xsT, dinv, w1c, b1c, Wih, Whh, bic, bhc, w2r)

    b2p = _make_scalar_agg_kernel(NP, n_rows)(
        ysc.reshape(NP), src2d, dst2d, zeros1).reshape(NC, NP)

    o = pl.pallas_call(
        _out_body,
        grid=(nblocks,),
        in_specs=[_row_spec(NC), _row_spec(1), _row_spec(1),
                  _full_spec((1, 1))],
        out_specs=_row_spec(1),
        out_shape=jax.ShapeDtypeStruct((1, NP), F32),
    )(b2p, ysc, dinv, b2s)

    return o[:, :N]
